# Initial kernel scaffold; baseline (speedup 1.0000x reference)
import functools

import jax
import jax.numpy as jnp
from jax import lax
from jax.experimental import pallas as pl
from jax.experimental.pallas import tpu as pltpu
from jax.experimental.pallas import tpu_sc as plsc

N_NODES = 10000
N_PAD = 10240
N_ROWS = 80
E = 320000
NC = 2
NS = 16
E_SC = E // NC
E_TILE = E_SC // NS
CHUNK = 125
NCHUNK = E_TILE // CHUNK
ROWS_TILE = N_ROWS // NS

_mesh = plsc.VectorSubcoreMesh(core_axis_name="c", subcore_axis_name="s")


@functools.partial(
    pl.kernel,
    mesh=_mesh,
    out_type=jax.ShapeDtypeStruct((NC, N_ROWS, 128), jnp.float32),
    scratch_types=[
        pltpu.VMEM((E_TILE,), jnp.int32),
        pltpu.VMEM((N_ROWS, 128), jnp.float32),
        pltpu.VMEM_SHARED((N_ROWS, 128), jnp.float32),
        pltpu.VMEM((1, N_ROWS), jnp.int32),
    ],
)
def _sc_degree(colf, zeros2d, idrows, out, col_v, local, acc, idr_v):
    c = lax.axis_index("c")
    s = lax.axis_index("s")
    pltpu.sync_copy(colf.at[c].at[s], col_v)
    pltpu.sync_copy(zeros2d, local)
    pltpu.sync_copy(idrows, idr_v)
    pltpu.sync_copy(zeros2d.at[pl.ds(s * ROWS_TILE, ROWS_TILE)],
                    acc.at[pl.ds(s * ROWS_TILE, ROWS_TILE)])
    ones = jnp.full((16,), 1.0, jnp.float32)

    def body(i, _):
        ic = col_v[pl.ds(i * 16, 16)]
        plsc.addupdate_scatter(local, [ic >> 7, ic & 127], ones)
        return 0

    lax.fori_loop(0, E_TILE // 16, body, 0)
    plsc.subcore_barrier()
    pltpu.sync_copy(local, acc.at[idr_v.at[0]], add=True)
    plsc.subcore_barrier()
    pltpu.sync_copy(acc.at[pl.ds(s * ROWS_TILE, ROWS_TILE)],
                    out.at[c].at[pl.ds(s * ROWS_TILE, ROWS_TILE)])


@functools.partial(
    pl.kernel,
    mesh=_mesh,
    out_type=jax.ShapeDtypeStruct((NC, N_PAD, 64), jnp.float32),
    scratch_types=[
        pltpu.VMEM((NCHUNK, CHUNK), jnp.int32),
        pltpu.VMEM((NCHUNK, CHUNK), jnp.int32),
        pltpu.VMEM((CHUNK, 64), jnp.float32),
        pltpu.VMEM((CHUNK, 64), jnp.float32),
        pltpu.VMEM_SHARED((N_PAD, 64), jnp.float32),
        pltpu.SemaphoreType.DMA,
        pltpu.SemaphoreType.DMA,
    ],
)
def _sc_agg64(row4, col4, g, zeros64, out, row_v, col_v, buf_a, buf_b, acc,
              sem_a, sem_b):
    c = lax.axis_index("c")
    s = lax.axis_index("s")
    pltpu.sync_copy(row4.at[c].at[s], row_v)
    pltpu.sync_copy(col4.at[c].at[s], col_v)
    nz = N_PAD // NS
    pltpu.sync_copy(zeros64.at[pl.ds(s * nz, nz)], acc.at[pl.ds(s * nz, nz)])
    plsc.subcore_barrier()

    pltpu.async_copy(g.at[row_v.at[0]], buf_a, sem_a)

    def body(j, _):
        pltpu.async_copy(g.at[row_v.at[2 * j + 1]], buf_b, sem_b)
        pltpu.make_async_copy(g.at[row_v.at[2 * j]], buf_a, sem_a).wait()
        pltpu.sync_copy(buf_a, acc.at[col_v.at[2 * j]], add=True)
        nj = jnp.minimum(2 * j + 2, NCHUNK - 1)
        pltpu.async_copy(g.at[row_v.at[nj]], buf_a, sem_a)
        pltpu.make_async_copy(g.at[row_v.at[2 * j + 1]], buf_b, sem_b).wait()
        pltpu.sync_copy(buf_b, acc.at[col_v.at[2 * j + 1]], add=True)
        return 0

    lax.fori_loop(0, NCHUNK // 2, body, 0)
    pltpu.make_async_copy(g.at[row_v.at[NCHUNK - 1]], buf_a, sem_a).wait()
    plsc.subcore_barrier()
    pltpu.sync_copy(acc.at[pl.ds(s * nz, nz)], out.at[c].at[pl.ds(s * nz, nz)])


@functools.partial(
    pl.kernel,
    mesh=_mesh,
    out_type=jax.ShapeDtypeStruct((NC, N_ROWS, 128), jnp.float32),
    scratch_types=[
        pltpu.VMEM((E_TILE,), jnp.int32),
        pltpu.VMEM((E_TILE,), jnp.int32),
        pltpu.VMEM((N_ROWS, 128), jnp.float32),
        pltpu.VMEM((N_ROWS, 128), jnp.float32),
        pltpu.VMEM_SHARED((N_ROWS, 128), jnp.float32),
        pltpu.VMEM((1, N_ROWS), jnp.int32),
    ],
)
def _sc_agg1(rowf, colf, g2d, zeros2d, idrows, out, row_v, col_v, g_local,
             local, acc, idr_v):
    c = lax.axis_index("c")
    s = lax.axis_index("s")
    pltpu.sync_copy(rowf.at[c].at[s], row_v)
    pltpu.sync_copy(colf.at[c].at[s], col_v)
    pltpu.sync_copy(g2d, g_local)
    pltpu.sync_copy(zeros2d, local)
    pltpu.sync_copy(idrows, idr_v)
    pltpu.sync_copy(zeros2d.at[pl.ds(s * ROWS_TILE, ROWS_TILE)],
                    acc.at[pl.ds(s * ROWS_TILE, ROWS_TILE)])

    def body(i, _):
        ir = row_v[pl.ds(i * 16, 16)]
        vals = plsc.load_gather(g_local, [ir >> 7, ir & 127])
        ic = col_v[pl.ds(i * 16, 16)]
        plsc.addupdate_scatter(local, [ic >> 7, ic & 127], vals)
        return 0

    lax.fori_loop(0, E_TILE // 16, body, 0)
    plsc.subcore_barrier()
    pltpu.sync_copy(local, acc.at[idr_v.at[0]], add=True)
    plsc.subcore_barrier()
    pltpu.sync_copy(acc.at[pl.ds(s * ROWS_TILE, ROWS_TILE)],
                    out.at[c].at[pl.ds(s * ROWS_TILE, ROWS_TILE)])


_R = 1024


def _tc_first(x, d, W):
    def body(d_ref, x_ref, w_ref, dinv_ref, g_ref):
        deg = 1.0 + d_ref[0] + d_ref[1]
        dinv = lax.rsqrt(deg)
        dinv_ref[...] = dinv
        g_ref[...] = dinv * jnp.dot(x_ref[...], w_ref[...],
                                    preferred_element_type=jnp.float32)

    return pl.pallas_call(
        body,
        grid=(N_PAD // _R,),
        in_specs=[
            pl.BlockSpec((NC, _R, 1), lambda i: (0, i, 0)),
            pl.BlockSpec((_R, 128), lambda i: (i, 0)),
            pl.BlockSpec((128, 64), lambda i: (0, 0)),
        ],
        out_specs=[
            pl.BlockSpec((_R, 1), lambda i: (i, 0)),
            pl.BlockSpec((_R, 64), lambda i: (i, 0)),
        ],
        out_shape=[
            jax.ShapeDtypeStruct((N_PAD, 1), jnp.float32),
            jax.ShapeDtypeStruct((N_PAD, 64), jnp.float32),
        ],
    )(d, x, W)


def _tc_layer(p, g, dinv, b, W):
    dout = W.shape[1]

    def body(p_ref, g_ref, dinv_ref, b_ref, w_ref, o_ref):
        dinv = dinv_ref[...]
        h = jnp.maximum(dinv * (p_ref[0] + p_ref[1] + g_ref[...]) + b_ref[...],
                        0.0)
        o_ref[...] = dinv * jnp.dot(h, w_ref[...],
                                    preferred_element_type=jnp.float32)

    return pl.pallas_call(
        body,
        grid=(N_PAD // _R,),
        in_specs=[
            pl.BlockSpec((NC, _R, 64), lambda i: (0, i, 0)),
            pl.BlockSpec((_R, 64), lambda i: (i, 0)),
            pl.BlockSpec((_R, 1), lambda i: (i, 0)),
            pl.BlockSpec((1, 64), lambda i: (0, 0)),
            pl.BlockSpec((64, dout), lambda i: (0, 0)),
        ],
        out_specs=pl.BlockSpec((_R, dout), lambda i: (i, 0)),
        out_shape=jax.ShapeDtypeStruct((N_PAD, dout), jnp.float32),
    )(p, g, dinv, b, W)


def _tc_final(q, g7, dinv, b_o):
    def body(q_ref, g_ref, dinv_ref, b_ref, o_ref):
        o_ref[...] = dinv_ref[...] * (q_ref[0] + q_ref[1] + g_ref[...]) \
            + b_ref[...]

    return pl.pallas_call(
        body,
        grid=(N_PAD // _R,),
        in_specs=[
            pl.BlockSpec((NC, _R, 1), lambda i: (0, i, 0)),
            pl.BlockSpec((_R, 1), lambda i: (i, 0)),
            pl.BlockSpec((_R, 1), lambda i: (i, 0)),
            pl.BlockSpec((1, 1), lambda i: (0, 0)),
        ],
        out_specs=pl.BlockSpec((_R, 1), lambda i: (i, 0)),
        out_shape=jax.ShapeDtypeStruct((N_PAD, 1), jnp.float32),
    )(q, g7, dinv, b_o)


def kernel(x, edge_index, W_i, b_i, Wh, bh, W_o, b_o):
    row = edge_index[0].astype(jnp.int32)
    col = edge_index[1].astype(jnp.int32)
    row4 = row.reshape(NC, NS, NCHUNK, CHUNK)
    col4 = col.reshape(NC, NS, NCHUNK, CHUNK)
    rowf = row.reshape(NC, NS, E_TILE)
    colf = col.reshape(NC, NS, E_TILE)

    zeros2d = jnp.zeros((N_ROWS, 128), jnp.float32)
    zeros64 = jnp.zeros((N_PAD, 64), jnp.float32)
    idrows = jnp.arange(N_ROWS, dtype=jnp.int32).reshape(1, N_ROWS)
    x_pad = jnp.zeros((N_PAD, 128), jnp.float32).at[:N_NODES].set(x)

    d = _sc_degree(colf, zeros2d, idrows).reshape(NC, N_PAD, 1)
    dinv, g = _tc_first(x_pad, d, W_i)

    biases = [b_i.reshape(1, 64)] + [bh[k].reshape(1, 64) for k in range(6)]
    weights = [Wh[k] for k in range(6)] + [W_o]
    for k in range(7):
        p = _sc_agg64(row4, col4, g, zeros64)
        g = _tc_layer(p, g, dinv, biases[k], weights[k])

    q = _sc_agg1(rowf, colf, g.reshape(N_ROWS, 128), zeros2d,
                 idrows).reshape(NC, N_PAD, 1)
    out = _tc_final(q, g, dinv, b_o)
    return out[:N_NODES]


# speedup vs baseline: 28.7096x; 28.7096x over previous
import functools

import jax
import jax.numpy as jnp
from jax import lax
from jax.experimental import pallas as pl
from jax.experimental.pallas import tpu as pltpu
from jax.experimental.pallas import tpu_sc as plsc

N_NODES = 10000
N_PAD = 10240
N_ROWS = 80
E = 320000
NC = 2
NS = 16
E_SC = E // NC
E_TILE = E_SC // NS
CHUNK = 125
NCHUNK = E_TILE // CHUNK
ROW_SLICE = 8
N_SLICERS = N_ROWS // ROW_SLICE

_mesh = plsc.VectorSubcoreMesh(core_axis_name="c", subcore_axis_name="s")
_sc_params = pltpu.CompilerParams(needs_layout_passes=False,
                                  use_tc_tiling_on_sc=False)


NZ = N_PAD // NS


@functools.partial(
    pl.kernel,
    mesh=_mesh,
    compiler_params=_sc_params,
    out_type=jax.ShapeDtypeStruct((NC, N_PAD), jnp.float32),
    scratch_types=[
        pltpu.VMEM((E_TILE,), jnp.int32),
        pltpu.VMEM((N_PAD,), jnp.float32),
        pltpu.VMEM((N_ROWS, 128), jnp.int32),
        pltpu.VMEM_SHARED((N_PAD,), jnp.float32),
    ],
)
def _sc_degree(colf, zeros1d, ident, out, col_v, local, idx_v, acc):
    c = lax.axis_index("c")
    s = lax.axis_index("s")
    pltpu.sync_copy(colf.at[c].at[s], col_v)
    pltpu.sync_copy(zeros1d, local)
    pltpu.sync_copy(ident, idx_v)
    pltpu.sync_copy(zeros1d.at[pl.ds(s * NZ, NZ)], acc.at[pl.ds(s * NZ, NZ)])
    ones = jnp.full((16,), 1.0, jnp.float32)

    def body(i, _):
        ic = col_v[pl.ds(i * 16, 16)]
        plsc.addupdate_scatter(local, [ic], ones)
        return 0

    lax.fori_loop(0, E_TILE // 16, body, 0)
    plsc.subcore_barrier()

    def red(j, _):
        pltpu.sync_copy(local.at[pl.ds(j * 128, 128)],
                        acc.at[idx_v.at[j]], add=True)
        return 0

    lax.fori_loop(0, N_ROWS, red, 0)
    plsc.subcore_barrier()
    pltpu.sync_copy(acc.at[pl.ds(s * NZ, NZ)], out.at[c].at[pl.ds(s * NZ, NZ)])


@functools.partial(
    pl.kernel,
    mesh=_mesh,
    compiler_params=_sc_params,
    out_type=jax.ShapeDtypeStruct((NC, N_PAD, 64), jnp.float32),
    scratch_types=[
        pltpu.VMEM((NCHUNK, CHUNK), jnp.int32),
        pltpu.VMEM((NCHUNK, CHUNK), jnp.int32),
        pltpu.VMEM((CHUNK, 64), jnp.float32),
        pltpu.VMEM((CHUNK, 64), jnp.float32),
        pltpu.VMEM_SHARED((N_PAD, 64), jnp.float32),
        pltpu.SemaphoreType.DMA,
        pltpu.SemaphoreType.DMA,
    ],
)
def _sc_agg64(row4, col4, g, zeros64, out, row_v, col_v, buf_a, buf_b, acc,
              sem_a, sem_b):
    c = lax.axis_index("c")
    s = lax.axis_index("s")
    pltpu.sync_copy(row4.at[c].at[s], row_v)
    pltpu.sync_copy(col4.at[c].at[s], col_v)
    nz = N_PAD // NS
    pltpu.sync_copy(zeros64.at[pl.ds(s * nz, nz)], acc.at[pl.ds(s * nz, nz)])
    plsc.subcore_barrier()

    pltpu.async_copy(g.at[row_v.at[0]], buf_a, sem_a)

    def body(j, _):
        pltpu.async_copy(g.at[row_v.at[2 * j + 1]], buf_b, sem_b)
        pltpu.make_async_copy(g.at[row_v.at[2 * j]], buf_a, sem_a).wait()
        pltpu.sync_copy(buf_a, acc.at[col_v.at[2 * j]], add=True)
        nj = jnp.minimum(2 * j + 2, NCHUNK - 1)
        pltpu.async_copy(g.at[row_v.at[nj]], buf_a, sem_a)
        pltpu.make_async_copy(g.at[row_v.at[2 * j + 1]], buf_b, sem_b).wait()
        pltpu.sync_copy(buf_b, acc.at[col_v.at[2 * j + 1]], add=True)
        return 0

    lax.fori_loop(0, NCHUNK // 2, body, 0)
    pltpu.make_async_copy(g.at[row_v.at[NCHUNK - 1]], buf_a, sem_a).wait()
    plsc.subcore_barrier()
    pltpu.sync_copy(acc.at[pl.ds(s * nz, nz)], out.at[c].at[pl.ds(s * nz, nz)])


@functools.partial(
    pl.kernel,
    mesh=_mesh,
    compiler_params=_sc_params,
    out_type=jax.ShapeDtypeStruct((NC, N_PAD), jnp.float32),
    scratch_types=[
        pltpu.VMEM((E_TILE,), jnp.int32),
        pltpu.VMEM((E_TILE,), jnp.int32),
        pltpu.VMEM((N_PAD,), jnp.float32),
        pltpu.VMEM((N_PAD,), jnp.float32),
        pltpu.VMEM((N_ROWS, 128), jnp.int32),
        pltpu.VMEM_SHARED((N_PAD,), jnp.float32),
    ],
)
def _sc_agg1(rowf, colf, g1d, zeros1d, ident, out, row_v, col_v, g_local,
             local, idx_v, acc):
    c = lax.axis_index("c")
    s = lax.axis_index("s")
    pltpu.sync_copy(rowf.at[c].at[s], row_v)
    pltpu.sync_copy(colf.at[c].at[s], col_v)
    pltpu.sync_copy(g1d, g_local)
    pltpu.sync_copy(zeros1d, local)
    pltpu.sync_copy(ident, idx_v)
    pltpu.sync_copy(zeros1d.at[pl.ds(s * NZ, NZ)], acc.at[pl.ds(s * NZ, NZ)])

    def body(i, _):
        ir = row_v[pl.ds(i * 16, 16)]
        vals = plsc.load_gather(g_local, [ir])
        ic = col_v[pl.ds(i * 16, 16)]
        plsc.addupdate_scatter(local, [ic], vals)
        return 0

    lax.fori_loop(0, E_TILE // 16, body, 0)
    plsc.subcore_barrier()

    def red(j, _):
        pltpu.sync_copy(local.at[pl.ds(j * 128, 128)],
                        acc.at[idx_v.at[j]], add=True)
        return 0

    lax.fori_loop(0, N_ROWS, red, 0)
    plsc.subcore_barrier()
    pltpu.sync_copy(acc.at[pl.ds(s * NZ, NZ)], out.at[c].at[pl.ds(s * NZ, NZ)])


_R = 1024


def _tc_first(x, d, W):
    def body(d_ref, x_ref, w_ref, dinv_ref, g_ref):
        deg = 1.0 + d_ref[0] + d_ref[1]
        dinv = lax.rsqrt(deg)
        dinv_ref[...] = dinv
        g_ref[...] = dinv * jnp.dot(x_ref[...], w_ref[...],
                                    preferred_element_type=jnp.float32)

    return pl.pallas_call(
        body,
        grid=(N_PAD // _R,),
        in_specs=[
            pl.BlockSpec((NC, _R, 1), lambda i: (0, i, 0)),
            pl.BlockSpec((_R, 128), lambda i: (i, 0)),
            pl.BlockSpec((128, 64), lambda i: (0, 0)),
        ],
        out_specs=[
            pl.BlockSpec((_R, 1), lambda i: (i, 0)),
            pl.BlockSpec((_R, 64), lambda i: (i, 0)),
        ],
        out_shape=[
            jax.ShapeDtypeStruct((N_PAD, 1), jnp.float32),
            jax.ShapeDtypeStruct((N_PAD, 64), jnp.float32),
        ],
    )(d, x, W)


def _tc_layer(p, g, dinv, b, W):
    dout = W.shape[1]

    def body(p_ref, g_ref, dinv_ref, b_ref, w_ref, o_ref):
        dinv = dinv_ref[...]
        h = jnp.maximum(dinv * (p_ref[0] + p_ref[1] + g_ref[...]) + b_ref[...],
                        0.0)
        o_ref[...] = dinv * jnp.dot(h, w_ref[...],
                                    preferred_element_type=jnp.float32)

    return pl.pallas_call(
        body,
        grid=(N_PAD // _R,),
        in_specs=[
            pl.BlockSpec((NC, _R, 64), lambda i: (0, i, 0)),
            pl.BlockSpec((_R, 64), lambda i: (i, 0)),
            pl.BlockSpec((_R, 1), lambda i: (i, 0)),
            pl.BlockSpec((1, 64), lambda i: (0, 0)),
            pl.BlockSpec((64, dout), lambda i: (0, 0)),
        ],
        out_specs=pl.BlockSpec((_R, dout), lambda i: (i, 0)),
        out_shape=jax.ShapeDtypeStruct((N_PAD, dout), jnp.float32),
    )(p, g, dinv, b, W)


def _tc_final(q, g7, dinv, b_o):
    def body(q_ref, g_ref, dinv_ref, b_ref, o_ref):
        o_ref[...] = dinv_ref[...] * (q_ref[0] + q_ref[1] + g_ref[...]) \
            + b_ref[...]

    return pl.pallas_call(
        body,
        grid=(N_PAD // _R,),
        in_specs=[
            pl.BlockSpec((NC, _R, 1), lambda i: (0, i, 0)),
            pl.BlockSpec((_R, 1), lambda i: (i, 0)),
            pl.BlockSpec((_R, 1), lambda i: (i, 0)),
            pl.BlockSpec((1, 1), lambda i: (0, 0)),
        ],
        out_specs=pl.BlockSpec((_R, 1), lambda i: (i, 0)),
        out_shape=jax.ShapeDtypeStruct((N_PAD, 1), jnp.float32),
    )(q, g7, dinv, b_o)


def kernel(x, edge_index, W_i, b_i, Wh, bh, W_o, b_o):
    row = edge_index[0].astype(jnp.int32)
    col = edge_index[1].astype(jnp.int32)
    row4 = row.reshape(NC, NS, NCHUNK, CHUNK)
    col4 = col.reshape(NC, NS, NCHUNK, CHUNK)
    rowf = row.reshape(NC, NS, E_TILE)
    colf = col.reshape(NC, NS, E_TILE)

    zeros1d = jnp.zeros((N_PAD,), jnp.float32)
    zeros64 = jnp.zeros((N_PAD, 64), jnp.float32)
    ident = jnp.arange(N_PAD, dtype=jnp.int32).reshape(N_ROWS, 128)
    x_pad = jnp.zeros((N_PAD, 128), jnp.float32).at[:N_NODES].set(x)

    d = _sc_degree(colf, zeros1d, ident).reshape(NC, N_PAD, 1)
    dinv, g = _tc_first(x_pad, d, W_i)

    biases = [b_i.reshape(1, 64)] + [bh[k].reshape(1, 64) for k in range(6)]
    weights = [Wh[k] for k in range(6)] + [W_o]
    for k in range(7):
        p = _sc_agg64(row4, col4, g, zeros64)
        g = _tc_layer(p, g, dinv, biases[k], weights[k])

    q = _sc_agg1(rowf, colf, g.reshape(N_PAD), zeros1d,
                 ident).reshape(NC, N_PAD, 1)
    out = _tc_final(q, g, dinv, b_o.reshape(1, 1))
    return out[:N_NODES]


# speedup vs baseline: 30.6490x; 1.0676x over previous
import functools

import jax
import jax.numpy as jnp
from jax import lax
from jax.experimental import pallas as pl
from jax.experimental.pallas import tpu as pltpu
from jax.experimental.pallas import tpu_sc as plsc

N_NODES = 10000
N_PAD = 10240
N_ROWS = 80
E = 320000
NC = 2
NS = 16
E_SC = E // NC
E_TILE = E_SC // NS
CHUNK = 125
NCHUNK = E_TILE // CHUNK
ROW_SLICE = 8
N_SLICERS = N_ROWS // ROW_SLICE

_mesh = plsc.VectorSubcoreMesh(core_axis_name="c", subcore_axis_name="s")
_sc_params = pltpu.CompilerParams(needs_layout_passes=False,
                                  use_tc_tiling_on_sc=False)


NZ = N_PAD // NS


@functools.partial(
    pl.kernel,
    mesh=_mesh,
    compiler_params=_sc_params,
    out_type=jax.ShapeDtypeStruct((NC, N_PAD), jnp.float32),
    scratch_types=[
        pltpu.VMEM((E_TILE,), jnp.int32),
        pltpu.VMEM((N_PAD,), jnp.float32),
        pltpu.VMEM((N_ROWS, 128), jnp.int32),
        pltpu.VMEM_SHARED((N_PAD,), jnp.float32),
    ],
)
def _sc_degree(colf, zeros1d, ident, out, col_v, local, idx_v, acc):
    c = lax.axis_index("c")
    s = lax.axis_index("s")
    pltpu.sync_copy(colf.at[c].at[s], col_v)
    pltpu.sync_copy(zeros1d, local)
    pltpu.sync_copy(ident, idx_v)
    pltpu.sync_copy(zeros1d.at[pl.ds(s * NZ, NZ)], acc.at[pl.ds(s * NZ, NZ)])
    ones = jnp.full((16,), 1.0, jnp.float32)

    def body(i, _):
        ic = col_v[pl.ds(i * 16, 16)]
        plsc.addupdate_scatter(local, [ic], ones)
        return 0

    lax.fori_loop(0, E_TILE // 16, body, 0)
    plsc.subcore_barrier()

    def red(j, _):
        pltpu.sync_copy(local.at[pl.ds(j * 128, 128)],
                        acc.at[idx_v.at[j]], add=True)
        return 0

    lax.fori_loop(0, N_ROWS, red, 0)
    plsc.subcore_barrier()
    pltpu.sync_copy(acc.at[pl.ds(s * NZ, NZ)], out.at[c].at[pl.ds(s * NZ, NZ)])


@functools.partial(
    pl.kernel,
    mesh=_mesh,
    compiler_params=_sc_params,
    out_type=jax.ShapeDtypeStruct((NC, N_PAD, 64), jnp.float32),
    scratch_types=[
        pltpu.VMEM((NCHUNK, CHUNK), jnp.int32),
        pltpu.VMEM((NCHUNK, CHUNK), jnp.int32),
        pltpu.VMEM((CHUNK, 64), jnp.float32),
        pltpu.VMEM((CHUNK, 64), jnp.float32),
        pltpu.VMEM((CHUNK, 64), jnp.float32),
        pltpu.VMEM((CHUNK, 64), jnp.float32),
        pltpu.VMEM_SHARED((N_PAD, 64), jnp.float32),
        pltpu.SemaphoreType.DMA,
        pltpu.SemaphoreType.DMA,
        pltpu.SemaphoreType.DMA,
        pltpu.SemaphoreType.DMA,
        pltpu.SemaphoreType.DMA,
        pltpu.SemaphoreType.DMA,
        pltpu.SemaphoreType.DMA,
        pltpu.SemaphoreType.DMA,
    ],
)
def _sc_agg64(row4, col4, g, zeros64, out, row_v, col_v, b0, b1, b2, b3, acc,
              g0, g1, g2, g3, s0, s1, s2, s3):
    c = lax.axis_index("c")
    s = lax.axis_index("s")
    pltpu.sync_copy(row4.at[c].at[s], row_v)
    pltpu.sync_copy(col4.at[c].at[s], col_v)
    nz = N_PAD // NS
    pltpu.sync_copy(zeros64.at[pl.ds(s * nz, nz)], acc.at[pl.ds(s * nz, nz)])
    plsc.subcore_barrier()

    bufs = (b0, b1, b2, b3)
    gsems = (g0, g1, g2, g3)
    ssems = (s0, s1, s2, s3)

    for k in range(4):
        pltpu.async_copy(g.at[row_v.at[k]], bufs[k], gsems[k])

    def body(j, _):
        for k in range(4):
            ch = 4 * j + k
            pltpu.make_async_copy(g.at[row_v.at[ch]], bufs[k], gsems[k]).wait()
            pltpu.async_copy(bufs[k], acc.at[col_v.at[ch]], ssems[k],
                             add=True)
        for k in range(4):
            nxt = jnp.minimum(4 * j + 4 + k, NCHUNK - 1)
            pltpu.make_async_copy(bufs[k], acc.at[col_v.at[4 * j + k]],
                                  ssems[k]).wait()
            pltpu.async_copy(g.at[row_v.at[nxt]], bufs[k], gsems[k])
        return 0

    lax.fori_loop(0, NCHUNK // 4 - 1, body, 0)
    for k in range(4):
        ch = NCHUNK - 4 + k
        pltpu.make_async_copy(g.at[row_v.at[ch]], bufs[k], gsems[k]).wait()
        pltpu.async_copy(bufs[k], acc.at[col_v.at[ch]], ssems[k], add=True)
    for k in range(4):
        pltpu.make_async_copy(bufs[k], acc.at[col_v.at[NCHUNK - 4 + k]],
                              ssems[k]).wait()
    plsc.subcore_barrier()
    pltpu.sync_copy(acc.at[pl.ds(s * nz, nz)], out.at[c].at[pl.ds(s * nz, nz)])


@functools.partial(
    pl.kernel,
    mesh=_mesh,
    compiler_params=_sc_params,
    out_type=jax.ShapeDtypeStruct((NC, N_PAD), jnp.float32),
    scratch_types=[
        pltpu.VMEM((E_TILE,), jnp.int32),
        pltpu.VMEM((E_TILE,), jnp.int32),
        pltpu.VMEM((N_PAD,), jnp.float32),
        pltpu.VMEM((N_PAD,), jnp.float32),
        pltpu.VMEM((N_ROWS, 128), jnp.int32),
        pltpu.VMEM_SHARED((N_PAD,), jnp.float32),
    ],
)
def _sc_agg1(rowf, colf, g1d, zeros1d, ident, out, row_v, col_v, g_local,
             local, idx_v, acc):
    c = lax.axis_index("c")
    s = lax.axis_index("s")
    pltpu.sync_copy(rowf.at[c].at[s], row_v)
    pltpu.sync_copy(colf.at[c].at[s], col_v)
    pltpu.sync_copy(g1d, g_local)
    pltpu.sync_copy(zeros1d, local)
    pltpu.sync_copy(ident, idx_v)
    pltpu.sync_copy(zeros1d.at[pl.ds(s * NZ, NZ)], acc.at[pl.ds(s * NZ, NZ)])

    def body(i, _):
        ir = row_v[pl.ds(i * 16, 16)]
        vals = plsc.load_gather(g_local, [ir])
        ic = col_v[pl.ds(i * 16, 16)]
        plsc.addupdate_scatter(local, [ic], vals)
        return 0

    lax.fori_loop(0, E_TILE // 16, body, 0)
    plsc.subcore_barrier()

    def red(j, _):
        pltpu.sync_copy(local.at[pl.ds(j * 128, 128)],
                        acc.at[idx_v.at[j]], add=True)
        return 0

    lax.fori_loop(0, N_ROWS, red, 0)
    plsc.subcore_barrier()
    pltpu.sync_copy(acc.at[pl.ds(s * NZ, NZ)], out.at[c].at[pl.ds(s * NZ, NZ)])


_R = 1024


def _tc_first(x, d, W):
    def body(d_ref, x_ref, w_ref, dinv_ref, g_ref):
        deg = 1.0 + d_ref[0] + d_ref[1]
        dinv = lax.rsqrt(deg)
        dinv_ref[...] = dinv
        g_ref[...] = dinv * jnp.dot(x_ref[...], w_ref[...],
                                    preferred_element_type=jnp.float32)

    return pl.pallas_call(
        body,
        grid=(N_PAD // _R,),
        in_specs=[
            pl.BlockSpec((NC, _R, 1), lambda i: (0, i, 0)),
            pl.BlockSpec((_R, 128), lambda i: (i, 0)),
            pl.BlockSpec((128, 64), lambda i: (0, 0)),
        ],
        out_specs=[
            pl.BlockSpec((_R, 1), lambda i: (i, 0)),
            pl.BlockSpec((_R, 64), lambda i: (i, 0)),
        ],
        out_shape=[
            jax.ShapeDtypeStruct((N_PAD, 1), jnp.float32),
            jax.ShapeDtypeStruct((N_PAD, 64), jnp.float32),
        ],
    )(d, x, W)


def _tc_layer(p, g, dinv, b, W):
    dout = W.shape[1]

    def body(p_ref, g_ref, dinv_ref, b_ref, w_ref, o_ref):
        dinv = dinv_ref[...]
        h = jnp.maximum(dinv * (p_ref[0] + p_ref[1] + g_ref[...]) + b_ref[...],
                        0.0)
        o_ref[...] = dinv * jnp.dot(h, w_ref[...],
                                    preferred_element_type=jnp.float32)

    return pl.pallas_call(
        body,
        grid=(N_PAD // _R,),
        in_specs=[
            pl.BlockSpec((NC, _R, 64), lambda i: (0, i, 0)),
            pl.BlockSpec((_R, 64), lambda i: (i, 0)),
            pl.BlockSpec((_R, 1), lambda i: (i, 0)),
            pl.BlockSpec((1, 64), lambda i: (0, 0)),
            pl.BlockSpec((64, dout), lambda i: (0, 0)),
        ],
        out_specs=pl.BlockSpec((_R, dout), lambda i: (i, 0)),
        out_shape=jax.ShapeDtypeStruct((N_PAD, dout), jnp.float32),
    )(p, g, dinv, b, W)


def _tc_final(q, g7, dinv, b_o):
    def body(q_ref, g_ref, dinv_ref, b_ref, o_ref):
        o_ref[...] = dinv_ref[...] * (q_ref[0] + q_ref[1] + g_ref[...]) \
            + b_ref[...]

    return pl.pallas_call(
        body,
        grid=(N_PAD // _R,),
        in_specs=[
            pl.BlockSpec((NC, _R, 1), lambda i: (0, i, 0)),
            pl.BlockSpec((_R, 1), lambda i: (i, 0)),
            pl.BlockSpec((_R, 1), lambda i: (i, 0)),
            pl.BlockSpec((1, 1), lambda i: (0, 0)),
        ],
        out_specs=pl.BlockSpec((_R, 1), lambda i: (i, 0)),
        out_shape=jax.ShapeDtypeStruct((N_PAD, 1), jnp.float32),
    )(q, g7, dinv, b_o)


def kernel(x, edge_index, W_i, b_i, Wh, bh, W_o, b_o):
    row = edge_index[0].astype(jnp.int32)
    col = edge_index[1].astype(jnp.int32)
    row4 = row.reshape(NC, NS, NCHUNK, CHUNK)
    col4 = col.reshape(NC, NS, NCHUNK, CHUNK)
    rowf = row.reshape(NC, NS, E_TILE)
    colf = col.reshape(NC, NS, E_TILE)

    zeros1d = jnp.zeros((N_PAD,), jnp.float32)
    zeros64 = jnp.zeros((N_PAD, 64), jnp.float32)
    ident = jnp.arange(N_PAD, dtype=jnp.int32).reshape(N_ROWS, 128)
    x_pad = jnp.zeros((N_PAD, 128), jnp.float32).at[:N_NODES].set(x)

    d = _sc_degree(colf, zeros1d, ident).reshape(NC, N_PAD, 1)
    dinv, g = _tc_first(x_pad, d, W_i)

    biases = [b_i.reshape(1, 64)] + [bh[k].reshape(1, 64) for k in range(6)]
    weights = [Wh[k] for k in range(6)] + [W_o]
    for k in range(7):
        p = _sc_agg64(row4, col4, g, zeros64)
        g = _tc_layer(p, g, dinv, biases[k], weights[k])

    q = _sc_agg1(rowf, colf, g.reshape(N_PAD), zeros1d,
                 ident).reshape(NC, N_PAD, 1)
    out = _tc_final(q, g, dinv, b_o.reshape(1, 1))
    return out[:N_NODES]


# speedup vs baseline: 34.8091x; 1.1357x over previous
import functools

import jax
import jax.numpy as jnp
from jax import lax
from jax.experimental import pallas as pl
from jax.experimental.pallas import tpu as pltpu
from jax.experimental.pallas import tpu_sc as plsc

N_NODES = 10000
N_PAD = 10240
N_ROWS = 80
E = 320000
NC = 2
NS = 16
E_SC = E // NC
E_TILE = E_SC // NS
CHUNK = 125
NCHUNK = E_TILE // CHUNK
ROW_SLICE = 8
N_SLICERS = N_ROWS // ROW_SLICE

_mesh = plsc.VectorSubcoreMesh(core_axis_name="c", subcore_axis_name="s")
_sc_params = pltpu.CompilerParams(needs_layout_passes=False,
                                  use_tc_tiling_on_sc=False)


NZ = N_PAD // NS


@functools.partial(
    pl.kernel,
    mesh=_mesh,
    compiler_params=_sc_params,
    out_type=[
        jax.ShapeDtypeStruct((NC, N_PAD), jnp.float32),
        jax.ShapeDtypeStruct((NC, N_PAD * 64), jnp.float32),
    ],
    scratch_types=[
        pltpu.VMEM((E_TILE,), jnp.int32),
        pltpu.VMEM((N_PAD,), jnp.float32),
        pltpu.VMEM((N_ROWS, 128), jnp.int32),
        pltpu.VMEM_SHARED((N_PAD,), jnp.float32),
        pltpu.VMEM((NZ,), jnp.float32),
        pltpu.VMEM((NZ * 64,), jnp.float32),
    ],
)
def _sc_degree(colf, zeros1d, ident, out, out_pk, col_v, local, idx_v, acc,
               tmp, local_pk):
    c = lax.axis_index("c")
    s = lax.axis_index("s")
    pltpu.sync_copy(colf.at[c].at[s], col_v)
    pltpu.sync_copy(zeros1d, local)
    pltpu.sync_copy(ident, idx_v)
    pltpu.sync_copy(zeros1d.at[pl.ds(s * NZ, NZ)], acc.at[pl.ds(s * NZ, NZ)])
    ones = jnp.full((16,), 1.0, jnp.float32)

    def body(i, _):
        ic = col_v[pl.ds(i * 16, 16)]
        plsc.addupdate_scatter(local, [ic], ones)
        return 0

    lax.fori_loop(0, E_TILE // 16, body, 0)
    plsc.subcore_barrier()

    def red(j, _):
        pltpu.sync_copy(local.at[pl.ds(j * 128, 128)],
                        acc.at[idx_v.at[j]], add=True)
        return 0

    lax.fori_loop(0, N_ROWS, red, 0)
    plsc.subcore_barrier()
    pltpu.sync_copy(acc.at[pl.ds(s * NZ, NZ)], out.at[c].at[pl.ds(s * NZ, NZ)])
    pltpu.sync_copy(acc.at[pl.ds(s * NZ, NZ)], tmp)
    lane = jax.lax.iota(jnp.int32, 16)

    def rep(i, _):
        v = tmp[pl.ds(i * 16, 16)]
        base = (i * 16 + lane) * 64
        for k in range(64):
            plsc.store_scatter(local_pk, [base + k], v)
        return 0

    lax.fori_loop(0, NZ // 16, rep, 0)
    pltpu.sync_copy(local_pk, out_pk.at[c].at[pl.ds(s * NZ * 64, NZ * 64)])


@functools.partial(
    pl.kernel,
    mesh=_mesh,
    compiler_params=_sc_params,
    out_type=jax.ShapeDtypeStruct((NC, N_PAD, 64), jnp.float32),
    scratch_types=[
        pltpu.VMEM((NCHUNK, CHUNK), jnp.int32),
        pltpu.VMEM((NCHUNK, CHUNK), jnp.int32),
        pltpu.VMEM((CHUNK, 64), jnp.float32),
        pltpu.VMEM((CHUNK, 64), jnp.float32),
        pltpu.VMEM((CHUNK, 64), jnp.float32),
        pltpu.VMEM((CHUNK, 64), jnp.float32),
        pltpu.VMEM_SHARED((N_PAD, 64), jnp.float32),
        pltpu.SemaphoreType.DMA,
        pltpu.SemaphoreType.DMA,
        pltpu.SemaphoreType.DMA,
        pltpu.SemaphoreType.DMA,
        pltpu.SemaphoreType.DMA,
        pltpu.SemaphoreType.DMA,
        pltpu.SemaphoreType.DMA,
        pltpu.SemaphoreType.DMA,
    ],
)
def _sc_agg64(row4, col4, g, zeros64, out, row_v, col_v, b0, b1, b2, b3, acc,
              g0, g1, g2, g3, s0, s1, s2, s3):
    c = lax.axis_index("c")
    s = lax.axis_index("s")
    pltpu.sync_copy(row4.at[c].at[s], row_v)
    pltpu.sync_copy(col4.at[c].at[s], col_v)
    nz = N_PAD // NS
    pltpu.sync_copy(zeros64.at[pl.ds(s * nz, nz)], acc.at[pl.ds(s * nz, nz)])
    plsc.subcore_barrier()

    bufs = (b0, b1, b2, b3)
    gsems = (g0, g1, g2, g3)
    ssems = (s0, s1, s2, s3)

    for k in range(4):
        pltpu.async_copy(g.at[row_v.at[k]], bufs[k], gsems[k])

    def body(j, _):
        for k in range(4):
            ch = 4 * j + k
            pltpu.make_async_copy(g.at[row_v.at[ch]], bufs[k], gsems[k]).wait()
            pltpu.async_copy(bufs[k], acc.at[col_v.at[ch]], ssems[k],
                             add=True)
        for k in range(4):
            nxt = jnp.minimum(4 * j + 4 + k, NCHUNK - 1)
            pltpu.make_async_copy(bufs[k], acc.at[col_v.at[4 * j + k]],
                                  ssems[k]).wait()
            pltpu.async_copy(g.at[row_v.at[nxt]], bufs[k], gsems[k])
        return 0

    lax.fori_loop(0, NCHUNK // 4 - 1, body, 0)
    for k in range(4):
        ch = NCHUNK - 4 + k
        pltpu.make_async_copy(g.at[row_v.at[ch]], bufs[k], gsems[k]).wait()
        pltpu.async_copy(bufs[k], acc.at[col_v.at[ch]], ssems[k], add=True)
    for k in range(4):
        pltpu.make_async_copy(bufs[k], acc.at[col_v.at[NCHUNK - 4 + k]],
                              ssems[k]).wait()
    plsc.subcore_barrier()
    pltpu.sync_copy(acc.at[pl.ds(s * nz, nz)], out.at[c].at[pl.ds(s * nz, nz)])


@functools.partial(
    pl.kernel,
    mesh=_mesh,
    compiler_params=_sc_params,
    out_type=jax.ShapeDtypeStruct((NC, N_PAD), jnp.float32),
    scratch_types=[
        pltpu.VMEM((E_TILE,), jnp.int32),
        pltpu.VMEM((E_TILE,), jnp.int32),
        pltpu.VMEM((N_PAD,), jnp.float32),
        pltpu.VMEM((N_PAD,), jnp.float32),
        pltpu.VMEM((N_ROWS, 128), jnp.int32),
        pltpu.VMEM_SHARED((N_PAD,), jnp.float32),
    ],
)
def _sc_agg1(rowf, colf, g1d, zeros1d, ident, out, row_v, col_v, g_local,
             local, idx_v, acc):
    c = lax.axis_index("c")
    s = lax.axis_index("s")
    pltpu.sync_copy(rowf.at[c].at[s], row_v)
    pltpu.sync_copy(colf.at[c].at[s], col_v)
    pltpu.sync_copy(g1d, g_local)
    pltpu.sync_copy(zeros1d, local)
    pltpu.sync_copy(ident, idx_v)
    pltpu.sync_copy(zeros1d.at[pl.ds(s * NZ, NZ)], acc.at[pl.ds(s * NZ, NZ)])

    def body(i, _):
        ir = row_v[pl.ds(i * 16, 16)]
        vals = plsc.load_gather(g_local, [ir])
        ic = col_v[pl.ds(i * 16, 16)]
        plsc.addupdate_scatter(local, [ic], vals)
        return 0

    lax.fori_loop(0, E_TILE // 16, body, 0)
    plsc.subcore_barrier()

    def red(j, _):
        pltpu.sync_copy(local.at[pl.ds(j * 128, 128)],
                        acc.at[idx_v.at[j]], add=True)
        return 0

    lax.fori_loop(0, N_ROWS, red, 0)
    plsc.subcore_barrier()
    pltpu.sync_copy(acc.at[pl.ds(s * NZ, NZ)], out.at[c].at[pl.ds(s * NZ, NZ)])


_R = 1024


_RP = _R // 2


def _tc_first(x_pk, dn, dpk, W2_i):
    def body(dn_ref, dpk_ref, x_ref, w_ref, dm1_ref, dpki_ref, g_ref):
        dm1_ref[...] = lax.rsqrt(1.0 + dn_ref[0] + dn_ref[1])
        dinv = lax.rsqrt(1.0 + dpk_ref[0] + dpk_ref[1])
        dpki_ref[...] = dinv
        g_ref[...] = dinv * jnp.dot(x_ref[...], w_ref[...],
                                    preferred_element_type=jnp.float32)

    return pl.pallas_call(
        body,
        grid=(N_PAD // _R,),
        in_specs=[
            pl.BlockSpec((NC, _R, 1), lambda i: (0, i, 0)),
            pl.BlockSpec((NC, _RP, 128), lambda i: (0, i, 0)),
            pl.BlockSpec((_RP, 256), lambda i: (i, 0)),
            pl.BlockSpec((256, 128), lambda i: (0, 0)),
        ],
        out_specs=[
            pl.BlockSpec((_R, 1), lambda i: (i, 0)),
            pl.BlockSpec((_RP, 128), lambda i: (i, 0)),
            pl.BlockSpec((_RP, 128), lambda i: (i, 0)),
        ],
        out_shape=[
            jax.ShapeDtypeStruct((N_PAD, 1), jnp.float32),
            jax.ShapeDtypeStruct((N_PAD // 2, 128), jnp.float32),
            jax.ShapeDtypeStruct((N_PAD // 2, 128), jnp.float32),
        ],
    )(dn, dpk, x_pk, W2_i)


def _tc_layer_pk(p_pk, g_pk, dinv_pk, b2, W2):
    def body(p_ref, g_ref, dinv_ref, b_ref, w_ref, o_ref):
        dinv = dinv_ref[...]
        h = jnp.maximum(dinv * (p_ref[0] + p_ref[1] + g_ref[...]) + b_ref[...],
                        0.0)
        o_ref[...] = dinv * jnp.dot(h, w_ref[...],
                                    preferred_element_type=jnp.float32)

    return pl.pallas_call(
        body,
        grid=(N_PAD // _R,),
        in_specs=[
            pl.BlockSpec((NC, _RP, 128), lambda i: (0, i, 0)),
            pl.BlockSpec((_RP, 128), lambda i: (i, 0)),
            pl.BlockSpec((_RP, 128), lambda i: (i, 0)),
            pl.BlockSpec((1, 128), lambda i: (0, 0)),
            pl.BlockSpec((128, 128), lambda i: (0, 0)),
        ],
        out_specs=pl.BlockSpec((_RP, 128), lambda i: (i, 0)),
        out_shape=jax.ShapeDtypeStruct((N_PAD // 2, 128), jnp.float32),
    )(p_pk, g_pk, dinv_pk, b2, W2)


def _tc_layer(p, g, dinv, b, W):
    dout = W.shape[1]

    def body(p_ref, g_ref, dinv_ref, b_ref, w_ref, o_ref):
        dinv = dinv_ref[...]
        h = jnp.maximum(dinv * (p_ref[0] + p_ref[1] + g_ref[...]) + b_ref[...],
                        0.0)
        o_ref[...] = dinv * jnp.dot(h, w_ref[...],
                                    preferred_element_type=jnp.float32)

    return pl.pallas_call(
        body,
        grid=(N_PAD // _R,),
        in_specs=[
            pl.BlockSpec((NC, _R, 64), lambda i: (0, i, 0)),
            pl.BlockSpec((_R, 64), lambda i: (i, 0)),
            pl.BlockSpec((_R, 1), lambda i: (i, 0)),
            pl.BlockSpec((1, 64), lambda i: (0, 0)),
            pl.BlockSpec((64, dout), lambda i: (0, 0)),
        ],
        out_specs=pl.BlockSpec((_R, dout), lambda i: (i, 0)),
        out_shape=jax.ShapeDtypeStruct((N_PAD, dout), jnp.float32),
    )(p, g, dinv, b, W)


def _tc_final(q, g7, dinv, b_o):
    def body(q_ref, g_ref, dinv_ref, b_ref, o_ref):
        o_ref[...] = dinv_ref[...] * (q_ref[0] + q_ref[1] + g_ref[...]) \
            + b_ref[...]

    return pl.pallas_call(
        body,
        grid=(N_PAD // _R,),
        in_specs=[
            pl.BlockSpec((NC, _R, 1), lambda i: (0, i, 0)),
            pl.BlockSpec((_R, 1), lambda i: (i, 0)),
            pl.BlockSpec((_R, 1), lambda i: (i, 0)),
            pl.BlockSpec((1, 1), lambda i: (0, 0)),
        ],
        out_specs=pl.BlockSpec((_R, 1), lambda i: (i, 0)),
        out_shape=jax.ShapeDtypeStruct((N_PAD, 1), jnp.float32),
    )(q, g7, dinv, b_o)


def kernel(x, edge_index, W_i, b_i, Wh, bh, W_o, b_o):
    row = edge_index[0].astype(jnp.int32)
    col = edge_index[1].astype(jnp.int32)
    row4 = row.reshape(NC, NS, NCHUNK, CHUNK)
    col4 = col.reshape(NC, NS, NCHUNK, CHUNK)
    rowf = row.reshape(NC, NS, E_TILE)
    colf = col.reshape(NC, NS, E_TILE)

    zeros1d = jnp.zeros((N_PAD,), jnp.float32)
    zeros64 = jnp.zeros((N_PAD, 64), jnp.float32)
    ident = jnp.arange(N_PAD, dtype=jnp.int32).reshape(N_ROWS, 128)
    x_pad = jnp.zeros((N_PAD, 128), jnp.float32).at[:N_NODES].set(x)

    def blockdiag2(W):
        din, dout = W.shape
        W2 = jnp.zeros((2 * din, 2 * dout), jnp.float32)
        return W2.at[:din, :dout].set(W).at[din:, dout:].set(W)

    dn_lin, dpk_flat = _sc_degree(colf, zeros1d, ident)
    dn = dn_lin.reshape(NC, N_PAD, 1)
    dpk = dpk_flat.reshape(NC, N_PAD // 2, 128)
    dinv_m1, dinv_pk, g = _tc_first(x_pad.reshape(N_PAD // 2, 256), dn, dpk,
                                    blockdiag2(W_i))

    for k in range(6):
        b = b_i if k == 0 else bh[k - 1]
        p = _sc_agg64(row4, col4, g.reshape(N_PAD, 64), zeros64)
        g = _tc_layer_pk(p.reshape(NC, N_PAD // 2, 128), g, dinv_pk,
                         jnp.concatenate([b, b]).reshape(1, 128),
                         blockdiag2(Wh[k]))

    p = _sc_agg64(row4, col4, g.reshape(N_PAD, 64), zeros64)
    g7 = _tc_layer(p, g.reshape(N_PAD, 64), dinv_m1, bh[5].reshape(1, 64),
                   W_o)
    q = _sc_agg1(rowf, colf, g7.reshape(N_PAD), zeros1d,
                 ident).reshape(NC, N_PAD, 1)
    out = _tc_final(q, g7, dinv_m1, b_o.reshape(1, 1))
    return out[:N_NODES]


# speedup vs baseline: 37.2899x; 1.0713x over previous
import functools

import jax
import jax.numpy as jnp
from jax import lax
from jax.experimental import pallas as pl
from jax.experimental.pallas import tpu as pltpu
from jax.experimental.pallas import tpu_sc as plsc

N_NODES = 10000
N_PAD = 10240
N_ROWS = 80
E = 320000
NC = 2
NS = 16
E_SC = E // NC
E_TILE = E_SC // NS
CHUNK = 125
NCHUNK = E_TILE // CHUNK
ROW_SLICE = 8
N_SLICERS = N_ROWS // ROW_SLICE

_mesh = plsc.VectorSubcoreMesh(core_axis_name="c", subcore_axis_name="s")
_sc_params = pltpu.CompilerParams(needs_layout_passes=False,
                                  use_tc_tiling_on_sc=False)


NZ = N_PAD // NS


@functools.partial(
    pl.kernel,
    mesh=_mesh,
    compiler_params=_sc_params,
    out_type=[
        jax.ShapeDtypeStruct((NC, N_PAD), jnp.float32),
        jax.ShapeDtypeStruct((NC, N_PAD * 64), jnp.float32),
    ],
    scratch_types=[
        pltpu.VMEM((E_TILE,), jnp.int32),
        pltpu.VMEM((N_PAD,), jnp.float32),
        pltpu.VMEM((N_ROWS, 128), jnp.int32),
        pltpu.VMEM_SHARED((N_PAD,), jnp.float32),
        pltpu.VMEM((NZ,), jnp.float32),
        pltpu.VMEM((NZ * 64,), jnp.float32),
        pltpu.SemaphoreType.DMA,
    ],
)
def _sc_degree(colf, zeros1d, ident, out, out_pk, col_v, local, idx_v, acc,
               tmp, local_pk, sem_r):
    c = lax.axis_index("c")
    s = lax.axis_index("s")
    pltpu.sync_copy(colf.at[c].at[s], col_v)
    pltpu.sync_copy(zeros1d, local)
    pltpu.sync_copy(ident, idx_v)
    pltpu.sync_copy(zeros1d.at[pl.ds(s * NZ, NZ)], acc.at[pl.ds(s * NZ, NZ)])
    ones = jnp.full((16,), 1.0, jnp.float32)

    def body(i, _):
        for u in range(4):
            ic = col_v[pl.ds((4 * i + u) * 16, 16)]
            plsc.addupdate_scatter(local, [ic], ones)
        return 0

    lax.fori_loop(0, E_TILE // 64, body, 0)
    plsc.subcore_barrier()

    def red(j, _):
        pltpu.async_copy(local.at[pl.ds(j * 128, 128)],
                         acc.at[idx_v.at[j]], sem_r, add=True)
        return 0

    lax.fori_loop(0, N_ROWS, red, 0)

    def red_wait(j, _):
        pltpu.make_async_copy(local.at[pl.ds(j * 128, 128)],
                              acc.at[idx_v.at[j]], sem_r).wait()
        return 0

    lax.fori_loop(0, N_ROWS, red_wait, 0)
    plsc.subcore_barrier()
    pltpu.sync_copy(acc.at[pl.ds(s * NZ, NZ)], out.at[c].at[pl.ds(s * NZ, NZ)])
    pltpu.sync_copy(acc.at[pl.ds(s * NZ, NZ)], tmp)
    lane = jax.lax.iota(jnp.int32, 16)

    def rep(i, _):
        v = tmp[pl.ds(i * 16, 16)]
        base = (i * 16 + lane) * 64
        for k in range(64):
            plsc.store_scatter(local_pk, [base + k], v)
        return 0

    lax.fori_loop(0, NZ // 16, rep, 0)
    pltpu.sync_copy(local_pk, out_pk.at[c].at[pl.ds(s * NZ * 64, NZ * 64)])


_DEPTH = 8


@functools.partial(
    pl.kernel,
    mesh=_mesh,
    compiler_params=_sc_params,
    out_type=jax.ShapeDtypeStruct((NC, N_PAD, 64), jnp.float32),
    scratch_types=(
        [pltpu.VMEM((NCHUNK, CHUNK), jnp.int32)] * 2
        + [pltpu.VMEM((CHUNK, 64), jnp.float32)] * _DEPTH
        + [pltpu.VMEM_SHARED((N_PAD, 64), jnp.float32)]
        + [pltpu.SemaphoreType.DMA] * (2 * _DEPTH)
    ),
)
def _sc_agg64(row4, col4, g, zeros64, out, row_v, col_v, *rest):
    bufs = rest[:_DEPTH]
    acc = rest[_DEPTH]
    gsems = rest[_DEPTH + 1:2 * _DEPTH + 1]
    ssems = rest[2 * _DEPTH + 1:]
    c = lax.axis_index("c")
    s = lax.axis_index("s")
    pltpu.sync_copy(row4.at[c].at[s], row_v)
    pltpu.sync_copy(col4.at[c].at[s], col_v)
    nz = N_PAD // NS
    pltpu.sync_copy(zeros64.at[pl.ds(s * nz, nz)], acc.at[pl.ds(s * nz, nz)])
    plsc.subcore_barrier()

    for k in range(_DEPTH):
        pltpu.async_copy(g.at[row_v.at[k]], bufs[k], gsems[k])

    def body(j, _):
        for k in range(_DEPTH):
            ch = _DEPTH * j + k
            pltpu.make_async_copy(g.at[row_v.at[ch]], bufs[k], gsems[k]).wait()
            pltpu.async_copy(bufs[k], acc.at[col_v.at[ch]], ssems[k],
                             add=True)
        for k in range(_DEPTH):
            nxt = _DEPTH * j + _DEPTH + k
            pltpu.make_async_copy(bufs[k], acc.at[col_v.at[_DEPTH * j + k]],
                                  ssems[k]).wait()
            pltpu.async_copy(g.at[row_v.at[nxt]], bufs[k], gsems[k])
        return 0

    lax.fori_loop(0, NCHUNK // _DEPTH - 1, body, 0)
    for k in range(_DEPTH):
        ch = NCHUNK - _DEPTH + k
        pltpu.make_async_copy(g.at[row_v.at[ch]], bufs[k], gsems[k]).wait()
        pltpu.async_copy(bufs[k], acc.at[col_v.at[ch]], ssems[k], add=True)
    for k in range(_DEPTH):
        pltpu.make_async_copy(bufs[k], acc.at[col_v.at[NCHUNK - _DEPTH + k]],
                              ssems[k]).wait()
    plsc.subcore_barrier()
    pltpu.sync_copy(acc.at[pl.ds(s * nz, nz)], out.at[c].at[pl.ds(s * nz, nz)])


@functools.partial(
    pl.kernel,
    mesh=_mesh,
    compiler_params=_sc_params,
    out_type=jax.ShapeDtypeStruct((NC, N_PAD), jnp.float32),
    scratch_types=[
        pltpu.VMEM((E_TILE,), jnp.int32),
        pltpu.VMEM((E_TILE,), jnp.int32),
        pltpu.VMEM((N_PAD,), jnp.float32),
        pltpu.VMEM((N_PAD,), jnp.float32),
        pltpu.VMEM((N_ROWS, 128), jnp.int32),
        pltpu.VMEM_SHARED((N_PAD,), jnp.float32),
        pltpu.SemaphoreType.DMA,
    ],
)
def _sc_agg1(rowf, colf, g1d, zeros1d, ident, out, row_v, col_v, g_local,
             local, idx_v, acc, sem_r):
    c = lax.axis_index("c")
    s = lax.axis_index("s")
    pltpu.sync_copy(rowf.at[c].at[s], row_v)
    pltpu.sync_copy(colf.at[c].at[s], col_v)
    pltpu.sync_copy(g1d, g_local)
    pltpu.sync_copy(zeros1d, local)
    pltpu.sync_copy(ident, idx_v)
    pltpu.sync_copy(zeros1d.at[pl.ds(s * NZ, NZ)], acc.at[pl.ds(s * NZ, NZ)])

    def body(i, _):
        for u in range(4):
            ir = row_v[pl.ds((4 * i + u) * 16, 16)]
            vals = plsc.load_gather(g_local, [ir])
            ic = col_v[pl.ds((4 * i + u) * 16, 16)]
            plsc.addupdate_scatter(local, [ic], vals)
        return 0

    lax.fori_loop(0, E_TILE // 64, body, 0)
    plsc.subcore_barrier()

    def red(j, _):
        pltpu.async_copy(local.at[pl.ds(j * 128, 128)],
                         acc.at[idx_v.at[j]], sem_r, add=True)
        return 0

    lax.fori_loop(0, N_ROWS, red, 0)

    def red_wait(j, _):
        pltpu.make_async_copy(local.at[pl.ds(j * 128, 128)],
                              acc.at[idx_v.at[j]], sem_r).wait()
        return 0

    lax.fori_loop(0, N_ROWS, red_wait, 0)
    plsc.subcore_barrier()
    pltpu.sync_copy(acc.at[pl.ds(s * NZ, NZ)], out.at[c].at[pl.ds(s * NZ, NZ)])


_R = 1024


_RP = _R // 2


def _tc_first(x_pk, dpk, W2_i):
    def body(dpk_ref, x_ref, w_ref, dpki_ref, g_ref):
        dinv = lax.rsqrt(1.0 + dpk_ref[0] + dpk_ref[1])
        dpki_ref[...] = dinv
        g_ref[...] = dinv * jnp.dot(x_ref[...], w_ref[...],
                                    preferred_element_type=jnp.float32)

    return pl.pallas_call(
        body,
        grid=(N_PAD // _R,),
        in_specs=[
            pl.BlockSpec((NC, _RP, 128), lambda i: (0, i, 0)),
            pl.BlockSpec((_RP, 256), lambda i: (i, 0)),
            pl.BlockSpec((256, 128), lambda i: (0, 0)),
        ],
        out_specs=[
            pl.BlockSpec((_RP, 128), lambda i: (i, 0)),
            pl.BlockSpec((_RP, 128), lambda i: (i, 0)),
        ],
        out_shape=[
            jax.ShapeDtypeStruct((N_PAD // 2, 128), jnp.float32),
            jax.ShapeDtypeStruct((N_PAD // 2, 128), jnp.float32),
        ],
    )(dpk, x_pk, W2_i)


def _tc_dinv_m1(dn):
    def body(dn_ref, dm1_ref):
        dm1_ref[...] = lax.rsqrt(1.0 + dn_ref[0] + dn_ref[1])

    return pl.pallas_call(
        body,
        grid=(N_PAD // _R,),
        in_specs=[pl.BlockSpec((NC, _R, 1), lambda i: (0, i, 0))],
        out_specs=pl.BlockSpec((_R, 1), lambda i: (i, 0)),
        out_shape=jax.ShapeDtypeStruct((N_PAD, 1), jnp.float32),
    )(dn)


def _tc_layer_pk(p_pk, g_pk, dinv_pk, b2, W2):
    def body(p_ref, g_ref, dinv_ref, b_ref, w_ref, o_ref):
        dinv = dinv_ref[...]
        h = jnp.maximum(dinv * (p_ref[0] + p_ref[1] + g_ref[...]) + b_ref[...],
                        0.0)
        o_ref[...] = dinv * jnp.dot(h, w_ref[...],
                                    preferred_element_type=jnp.float32)

    return pl.pallas_call(
        body,
        grid=(N_PAD // _R,),
        in_specs=[
            pl.BlockSpec((NC, _RP, 128), lambda i: (0, i, 0)),
            pl.BlockSpec((_RP, 128), lambda i: (i, 0)),
            pl.BlockSpec((_RP, 128), lambda i: (i, 0)),
            pl.BlockSpec((1, 128), lambda i: (0, 0)),
            pl.BlockSpec((128, 128), lambda i: (0, 0)),
        ],
        out_specs=pl.BlockSpec((_RP, 128), lambda i: (i, 0)),
        out_shape=jax.ShapeDtypeStruct((N_PAD // 2, 128), jnp.float32),
    )(p_pk, g_pk, dinv_pk, b2, W2)


def _tc_layer(p, g, dinv, b, W):
    dout = W.shape[1]

    def body(p_ref, g_ref, dinv_ref, b_ref, w_ref, o_ref):
        dinv = dinv_ref[...]
        h = jnp.maximum(dinv * (p_ref[0] + p_ref[1] + g_ref[...]) + b_ref[...],
                        0.0)
        o_ref[...] = dinv * jnp.dot(h, w_ref[...],
                                    preferred_element_type=jnp.float32)

    return pl.pallas_call(
        body,
        grid=(N_PAD // _R,),
        in_specs=[
            pl.BlockSpec((NC, _R, 64), lambda i: (0, i, 0)),
            pl.BlockSpec((_R, 64), lambda i: (i, 0)),
            pl.BlockSpec((_R, 1), lambda i: (i, 0)),
            pl.BlockSpec((1, 64), lambda i: (0, 0)),
            pl.BlockSpec((64, dout), lambda i: (0, 0)),
        ],
        out_specs=pl.BlockSpec((_R, dout), lambda i: (i, 0)),
        out_shape=jax.ShapeDtypeStruct((N_PAD, dout), jnp.float32),
    )(p, g, dinv, b, W)


def _tc_final(q, g7, dinv, b_o):
    def body(q_ref, g_ref, dinv_ref, b_ref, o_ref):
        o_ref[...] = dinv_ref[...] * (q_ref[0] + q_ref[1] + g_ref[...]) \
            + b_ref[...]

    return pl.pallas_call(
        body,
        grid=(N_PAD // _R,),
        in_specs=[
            pl.BlockSpec((NC, _R, 1), lambda i: (0, i, 0)),
            pl.BlockSpec((_R, 1), lambda i: (i, 0)),
            pl.BlockSpec((_R, 1), lambda i: (i, 0)),
            pl.BlockSpec((1, 1), lambda i: (0, 0)),
        ],
        out_specs=pl.BlockSpec((_R, 1), lambda i: (i, 0)),
        out_shape=jax.ShapeDtypeStruct((N_PAD, 1), jnp.float32),
    )(q, g7, dinv, b_o)


def kernel(x, edge_index, W_i, b_i, Wh, bh, W_o, b_o):
    row = edge_index[0].astype(jnp.int32)
    col = edge_index[1].astype(jnp.int32)
    row4 = row.reshape(NC, NS, NCHUNK, CHUNK)
    col4 = col.reshape(NC, NS, NCHUNK, CHUNK)
    rowf = row.reshape(NC, NS, E_TILE)
    colf = col.reshape(NC, NS, E_TILE)

    zeros1d = jnp.zeros((N_PAD,), jnp.float32)
    zeros64 = jnp.zeros((N_PAD, 64), jnp.float32)
    ident = jnp.arange(N_PAD, dtype=jnp.int32).reshape(N_ROWS, 128)
    x_pad = jnp.zeros((N_PAD, 128), jnp.float32).at[:N_NODES].set(x)

    def blockdiag2(W):
        din, dout = W.shape
        W2 = jnp.zeros((2 * din, 2 * dout), jnp.float32)
        return W2.at[:din, :dout].set(W).at[din:, dout:].set(W)

    dn_lin, dpk_flat = _sc_degree(colf, zeros1d, ident)
    dpk = dpk_flat.reshape(NC, N_PAD // 2, 128)
    dinv_pk, g = _tc_first(x_pad.reshape(N_PAD // 2, 256), dpk,
                           blockdiag2(W_i))

    for k in range(6):
        b = b_i if k == 0 else bh[k - 1]
        p = _sc_agg64(row4, col4, g.reshape(N_PAD, 64), zeros64)
        g = _tc_layer_pk(p.reshape(NC, N_PAD // 2, 128), g, dinv_pk,
                         jnp.concatenate([b, b]).reshape(1, 128),
                         blockdiag2(Wh[k]))

    dinv_m1 = _tc_dinv_m1(dn_lin.reshape(NC, N_PAD, 1))
    p = _sc_agg64(row4, col4, g.reshape(N_PAD, 64), zeros64)
    g7 = _tc_layer(p, g.reshape(N_PAD, 64), dinv_m1, bh[5].reshape(1, 64),
                   W_o)
    q = _sc_agg1(rowf, colf, g7.reshape(N_PAD), zeros1d,
                 ident).reshape(NC, N_PAD, 1)
    out = _tc_final(q, g7, dinv_m1, b_o.reshape(1, 1))
    return out[:N_NODES]


# speedup vs baseline: 37.3224x; 1.0009x over previous
import functools

import jax
import jax.numpy as jnp
from jax import lax
from jax.experimental import pallas as pl
from jax.experimental.pallas import tpu as pltpu
from jax.experimental.pallas import tpu_sc as plsc

N_NODES = 10000
N_PAD = 10240
N_ROWS = 80
E = 320000
NC = 2
NS = 16
E_SC = E // NC
E_TILE = E_SC // NS
CHUNK = 125
NCHUNK = E_TILE // CHUNK
ROW_SLICE = 8
N_SLICERS = N_ROWS // ROW_SLICE

_mesh = plsc.VectorSubcoreMesh(core_axis_name="c", subcore_axis_name="s")
_sc_params = pltpu.CompilerParams(needs_layout_passes=False,
                                  use_tc_tiling_on_sc=False)


NZ = N_PAD // NS


@functools.partial(
    pl.kernel,
    mesh=_mesh,
    compiler_params=_sc_params,
    out_type=[
        jax.ShapeDtypeStruct((NC, N_PAD), jnp.float32),
        jax.ShapeDtypeStruct((NC, N_PAD * 64), jnp.float32),
    ],
    scratch_types=[
        pltpu.VMEM((E_TILE,), jnp.int32),
        pltpu.VMEM((N_PAD,), jnp.float32),
        pltpu.VMEM((N_ROWS, 128), jnp.int32),
        pltpu.VMEM_SHARED((N_PAD,), jnp.float32),
        pltpu.VMEM((NZ,), jnp.float32),
        pltpu.VMEM((NZ * 64,), jnp.float32),
        pltpu.SemaphoreType.DMA,
    ],
)
def _sc_degree(colf, zeros1d, ident, out, out_pk, col_v, local, idx_v, acc,
               tmp, local_pk, sem_r):
    c = lax.axis_index("c")
    s = lax.axis_index("s")
    pltpu.sync_copy(colf.at[c].at[s], col_v)
    pltpu.sync_copy(zeros1d, local)
    pltpu.sync_copy(ident, idx_v)
    pltpu.sync_copy(zeros1d.at[pl.ds(s * NZ, NZ)], acc.at[pl.ds(s * NZ, NZ)])
    ones = jnp.full((16,), 1.0, jnp.float32)

    def body(i, _):
        ic = col_v[pl.ds(i * 16, 16)]
        plsc.addupdate_scatter(local, [ic], ones)
        return 0

    lax.fori_loop(0, E_TILE // 16, body, 0)
    plsc.subcore_barrier()

    def red(j, _):
        pltpu.async_copy(local.at[pl.ds(j * 128, 128)],
                         acc.at[idx_v.at[j]], sem_r, add=True)
        return 0

    lax.fori_loop(0, N_ROWS, red, 0)

    def red_wait(j, _):
        pltpu.make_async_copy(local.at[pl.ds(j * 128, 128)],
                              acc.at[idx_v.at[j]], sem_r).wait()
        return 0

    lax.fori_loop(0, N_ROWS, red_wait, 0)
    plsc.subcore_barrier()
    pltpu.sync_copy(acc.at[pl.ds(s * NZ, NZ)], out.at[c].at[pl.ds(s * NZ, NZ)])
    pltpu.sync_copy(acc.at[pl.ds(s * NZ, NZ)], tmp)
    lane = jax.lax.iota(jnp.int32, 16)

    def rep(i, _):
        v = tmp[pl.ds(i * 16, 16)]
        base = (i * 16 + lane) * 64
        for k in range(64):
            plsc.store_scatter(local_pk, [base + k], v)
        return 0

    lax.fori_loop(0, NZ // 16, rep, 0)
    pltpu.sync_copy(local_pk, out_pk.at[c].at[pl.ds(s * NZ * 64, NZ * 64)])


_DEPTH = 8


@functools.partial(
    pl.kernel,
    mesh=_mesh,
    compiler_params=_sc_params,
    out_type=jax.ShapeDtypeStruct((NC, N_PAD, 64), jnp.float32),
    scratch_types=(
        [pltpu.VMEM((NCHUNK, CHUNK), jnp.int32)] * 2
        + [pltpu.VMEM((CHUNK, 64), jnp.float32)] * _DEPTH
        + [pltpu.VMEM_SHARED((N_PAD, 64), jnp.float32)]
        + [pltpu.SemaphoreType.DMA] * (2 * _DEPTH)
    ),
)
def _sc_agg64(row4, col4, g, zeros64, out, row_v, col_v, *rest):
    bufs = rest[:_DEPTH]
    acc = rest[_DEPTH]
    gsems = rest[_DEPTH + 1:2 * _DEPTH + 1]
    ssems = rest[2 * _DEPTH + 1:]
    c = lax.axis_index("c")
    s = lax.axis_index("s")
    pltpu.sync_copy(row4.at[c].at[s], row_v)
    pltpu.sync_copy(col4.at[c].at[s], col_v)
    nz = N_PAD // NS
    pltpu.sync_copy(zeros64.at[pl.ds(s * nz, nz)], acc.at[pl.ds(s * nz, nz)])
    plsc.subcore_barrier()

    for k in range(_DEPTH):
        pltpu.async_copy(g.at[row_v.at[k]], bufs[k], gsems[k])

    def body(j, _):
        for k in range(_DEPTH):
            ch = _DEPTH * j + k
            pltpu.make_async_copy(g.at[row_v.at[ch]], bufs[k], gsems[k]).wait()
            pltpu.async_copy(bufs[k], acc.at[col_v.at[ch]], ssems[k],
                             add=True)
        for k in range(_DEPTH):
            nxt = _DEPTH * j + _DEPTH + k
            pltpu.make_async_copy(bufs[k], acc.at[col_v.at[_DEPTH * j + k]],
                                  ssems[k]).wait()
            pltpu.async_copy(g.at[row_v.at[nxt]], bufs[k], gsems[k])
        return 0

    lax.fori_loop(0, NCHUNK // _DEPTH - 1, body, 0)
    for k in range(_DEPTH):
        ch = NCHUNK - _DEPTH + k
        pltpu.make_async_copy(g.at[row_v.at[ch]], bufs[k], gsems[k]).wait()
        pltpu.async_copy(bufs[k], acc.at[col_v.at[ch]], ssems[k], add=True)
    for k in range(_DEPTH):
        pltpu.make_async_copy(bufs[k], acc.at[col_v.at[NCHUNK - _DEPTH + k]],
                              ssems[k]).wait()
    plsc.subcore_barrier()
    pltpu.sync_copy(acc.at[pl.ds(s * nz, nz)], out.at[c].at[pl.ds(s * nz, nz)])


@functools.partial(
    pl.kernel,
    mesh=_mesh,
    compiler_params=_sc_params,
    out_type=jax.ShapeDtypeStruct((NC, N_PAD), jnp.float32),
    scratch_types=[
        pltpu.VMEM((E_TILE,), jnp.int32),
        pltpu.VMEM((E_TILE,), jnp.int32),
        pltpu.VMEM((N_PAD,), jnp.float32),
        pltpu.VMEM((N_PAD,), jnp.float32),
        pltpu.VMEM((N_ROWS, 128), jnp.int32),
        pltpu.VMEM_SHARED((N_PAD,), jnp.float32),
        pltpu.SemaphoreType.DMA,
    ],
)
def _sc_agg1(rowf, colf, g1d, zeros1d, ident, out, row_v, col_v, g_local,
             local, idx_v, acc, sem_r):
    c = lax.axis_index("c")
    s = lax.axis_index("s")
    pltpu.sync_copy(rowf.at[c].at[s], row_v)
    pltpu.sync_copy(colf.at[c].at[s], col_v)
    pltpu.sync_copy(g1d, g_local)
    pltpu.sync_copy(zeros1d, local)
    pltpu.sync_copy(ident, idx_v)
    pltpu.sync_copy(zeros1d.at[pl.ds(s * NZ, NZ)], acc.at[pl.ds(s * NZ, NZ)])

    def body(i, _):
        ir = row_v[pl.ds(i * 16, 16)]
        vals = plsc.load_gather(g_local, [ir])
        ic = col_v[pl.ds(i * 16, 16)]
        plsc.addupdate_scatter(local, [ic], vals)
        return 0

    lax.fori_loop(0, E_TILE // 16, body, 0)
    plsc.subcore_barrier()

    def red(j, _):
        pltpu.async_copy(local.at[pl.ds(j * 128, 128)],
                         acc.at[idx_v.at[j]], sem_r, add=True)
        return 0

    lax.fori_loop(0, N_ROWS, red, 0)

    def red_wait(j, _):
        pltpu.make_async_copy(local.at[pl.ds(j * 128, 128)],
                              acc.at[idx_v.at[j]], sem_r).wait()
        return 0

    lax.fori_loop(0, N_ROWS, red_wait, 0)
    plsc.subcore_barrier()
    pltpu.sync_copy(acc.at[pl.ds(s * NZ, NZ)], out.at[c].at[pl.ds(s * NZ, NZ)])


_R = 1024


_RP = _R // 2


def _tc_first(x_pk, dpk, W2_i):
    def body(dpk_ref, x_ref, w_ref, dpki_ref, g_ref):
        dinv = lax.rsqrt(1.0 + dpk_ref[0] + dpk_ref[1])
        dpki_ref[...] = dinv
        g_ref[...] = dinv * jnp.dot(x_ref[...], w_ref[...],
                                    preferred_element_type=jnp.float32)

    return pl.pallas_call(
        body,
        grid=(N_PAD // _R,),
        in_specs=[
            pl.BlockSpec((NC, _RP, 128), lambda i: (0, i, 0)),
            pl.BlockSpec((_RP, 256), lambda i: (i, 0)),
            pl.BlockSpec((256, 128), lambda i: (0, 0)),
        ],
        out_specs=[
            pl.BlockSpec((_RP, 128), lambda i: (i, 0)),
            pl.BlockSpec((_RP, 128), lambda i: (i, 0)),
        ],
        out_shape=[
            jax.ShapeDtypeStruct((N_PAD // 2, 128), jnp.float32),
            jax.ShapeDtypeStruct((N_PAD // 2, 128), jnp.float32),
        ],
    )(dpk, x_pk, W2_i)


def _tc_dinv_m1(dn):
    def body(dn_ref, dm1_ref):
        dm1_ref[...] = lax.rsqrt(1.0 + dn_ref[0] + dn_ref[1])

    return pl.pallas_call(
        body,
        grid=(N_PAD // _R,),
        in_specs=[pl.BlockSpec((NC, _R, 1), lambda i: (0, i, 0))],
        out_specs=pl.BlockSpec((_R, 1), lambda i: (i, 0)),
        out_shape=jax.ShapeDtypeStruct((N_PAD, 1), jnp.float32),
    )(dn)


def _tc_layer_pk(p_pk, g_pk, dinv_pk, b2, W2):
    def body(p_ref, g_ref, dinv_ref, b_ref, w_ref, o_ref):
        dinv = dinv_ref[...]
        h = jnp.maximum(dinv * (p_ref[0] + p_ref[1] + g_ref[...]) + b_ref[...],
                        0.0)
        o_ref[...] = dinv * jnp.dot(h, w_ref[...],
                                    preferred_element_type=jnp.float32)

    return pl.pallas_call(
        body,
        grid=(N_PAD // _R,),
        in_specs=[
            pl.BlockSpec((NC, _RP, 128), lambda i: (0, i, 0)),
            pl.BlockSpec((_RP, 128), lambda i: (i, 0)),
            pl.BlockSpec((_RP, 128), lambda i: (i, 0)),
            pl.BlockSpec((1, 128), lambda i: (0, 0)),
            pl.BlockSpec((128, 128), lambda i: (0, 0)),
        ],
        out_specs=pl.BlockSpec((_RP, 128), lambda i: (i, 0)),
        out_shape=jax.ShapeDtypeStruct((N_PAD // 2, 128), jnp.float32),
    )(p_pk, g_pk, dinv_pk, b2, W2)


def _tc_layer(p, g, dinv, b, W):
    dout = W.shape[1]

    def body(p_ref, g_ref, dinv_ref, b_ref, w_ref, o_ref):
        dinv = dinv_ref[...]
        h = jnp.maximum(dinv * (p_ref[0] + p_ref[1] + g_ref[...]) + b_ref[...],
                        0.0)
        o_ref[...] = dinv * jnp.dot(h, w_ref[...],
                                    preferred_element_type=jnp.float32)

    return pl.pallas_call(
        body,
        grid=(N_PAD // _R,),
        in_specs=[
            pl.BlockSpec((NC, _R, 64), lambda i: (0, i, 0)),
            pl.BlockSpec((_R, 64), lambda i: (i, 0)),
            pl.BlockSpec((_R, 1), lambda i: (i, 0)),
            pl.BlockSpec((1, 64), lambda i: (0, 0)),
            pl.BlockSpec((64, dout), lambda i: (0, 0)),
        ],
        out_specs=pl.BlockSpec((_R, dout), lambda i: (i, 0)),
        out_shape=jax.ShapeDtypeStruct((N_PAD, dout), jnp.float32),
    )(p, g, dinv, b, W)


def _tc_final(q, g7, dinv, b_o):
    def body(q_ref, g_ref, dinv_ref, b_ref, o_ref):
        o_ref[...] = dinv_ref[...] * (q_ref[0] + q_ref[1] + g_ref[...]) \
            + b_ref[...]

    return pl.pallas_call(
        body,
        grid=(N_PAD // _R,),
        in_specs=[
            pl.BlockSpec((NC, _R, 1), lambda i: (0, i, 0)),
            pl.BlockSpec((_R, 1), lambda i: (i, 0)),
            pl.BlockSpec((_R, 1), lambda i: (i, 0)),
            pl.BlockSpec((1, 1), lambda i: (0, 0)),
        ],
        out_specs=pl.BlockSpec((_R, 1), lambda i: (i, 0)),
        out_shape=jax.ShapeDtypeStruct((N_PAD, 1), jnp.float32),
    )(q, g7, dinv, b_o)


def kernel(x, edge_index, W_i, b_i, Wh, bh, W_o, b_o):
    row = edge_index[0].astype(jnp.int32)
    col = edge_index[1].astype(jnp.int32)
    row4 = row.reshape(NC, NS, NCHUNK, CHUNK)
    col4 = col.reshape(NC, NS, NCHUNK, CHUNK)
    rowf = row.reshape(NC, NS, E_TILE)
    colf = col.reshape(NC, NS, E_TILE)

    zeros1d = jnp.zeros((N_PAD,), jnp.float32)
    zeros64 = jnp.zeros((N_PAD, 64), jnp.float32)
    ident = jnp.arange(N_PAD, dtype=jnp.int32).reshape(N_ROWS, 128)
    x_pad = jnp.zeros((N_PAD, 128), jnp.float32).at[:N_NODES].set(x)

    def blockdiag2(W):
        din, dout = W.shape
        W2 = jnp.zeros((2 * din, 2 * dout), jnp.float32)
        return W2.at[:din, :dout].set(W).at[din:, dout:].set(W)

    dn_lin, dpk_flat = _sc_degree(colf, zeros1d, ident)
    dpk = dpk_flat.reshape(NC, N_PAD // 2, 128)
    dinv_pk, g = _tc_first(x_pad.reshape(N_PAD // 2, 256), dpk,
                           blockdiag2(W_i))

    for k in range(6):
        b = b_i if k == 0 else bh[k - 1]
        p = _sc_agg64(row4, col4, g.reshape(N_PAD, 64), zeros64)
        g = _tc_layer_pk(p.reshape(NC, N_PAD // 2, 128), g, dinv_pk,
                         jnp.concatenate([b, b]).reshape(1, 128),
                         blockdiag2(Wh[k]))

    dinv_m1 = _tc_dinv_m1(dn_lin.reshape(NC, N_PAD, 1))
    p = _sc_agg64(row4, col4, g.reshape(N_PAD, 64), zeros64)
    g7 = _tc_layer(p, g.reshape(N_PAD, 64), dinv_m1, bh[5].reshape(1, 64),
                   W_o)
    q = _sc_agg1(rowf, colf, g7.reshape(N_PAD), zeros1d,
                 ident).reshape(NC, N_PAD, 1)
    out = _tc_final(q, g7, dinv_m1, b_o.reshape(1, 1))
    return out[:N_NODES]


# speedup vs baseline: 40.1642x; 1.0761x over previous
import functools

import jax
import jax.numpy as jnp
from jax import lax
from jax.experimental import pallas as pl
from jax.experimental.pallas import tpu as pltpu
from jax.experimental.pallas import tpu_sc as plsc

N_NODES = 10000
N_PAD = 10240
N_ROWS = 80
E = 320000
NC = 2
NS = 16
E_SC = E // NC
E_TILE = E_SC // NS
CHUNK = 125
NCHUNK = E_TILE // CHUNK
ROW_SLICE = 8
N_SLICERS = N_ROWS // ROW_SLICE

_mesh = plsc.VectorSubcoreMesh(core_axis_name="c", subcore_axis_name="s")
_sc_params = pltpu.CompilerParams(needs_layout_passes=False,
                                  use_tc_tiling_on_sc=False)


NZ = N_PAD // NS


@functools.partial(
    pl.kernel,
    mesh=_mesh,
    compiler_params=_sc_params,
    out_type=[
        jax.ShapeDtypeStruct((NC, N_PAD), jnp.float32),
        jax.ShapeDtypeStruct((NC, N_PAD * 64), jnp.float32),
    ],
    scratch_types=[
        pltpu.VMEM((E_TILE,), jnp.int32),
        pltpu.VMEM((N_PAD,), jnp.float32),
        pltpu.VMEM((N_PAD,), jnp.float32),
        pltpu.VMEM((N_ROWS, 128), jnp.int32),
        pltpu.VMEM_SHARED((N_PAD,), jnp.float32),
        pltpu.VMEM((NZ,), jnp.float32),
        pltpu.VMEM((NZ * 64,), jnp.float32),
        pltpu.SemaphoreType.DMA,
    ],
)
def _sc_degree(colf, zeros1d, ident, out, out_pk, col_v, loc_a, loc_b, idx_v,
               acc, tmp, local_pk, sem_r):
    c = lax.axis_index("c")
    s = lax.axis_index("s")
    pltpu.sync_copy(colf.at[c].at[s], col_v)
    pltpu.sync_copy(zeros1d, loc_a)
    pltpu.sync_copy(zeros1d, loc_b)
    pltpu.sync_copy(ident, idx_v)
    pltpu.sync_copy(zeros1d.at[pl.ds(s * NZ, NZ)], acc.at[pl.ds(s * NZ, NZ)])
    ones = jnp.full((16,), 1.0, jnp.float32)

    def body(i, _):
        ic = col_v[pl.ds(2 * i * 16, 16)]
        plsc.addupdate_scatter(loc_a, [ic], ones)
        ic2 = col_v[pl.ds((2 * i + 1) * 16, 16)]
        plsc.addupdate_scatter(loc_b, [ic2], ones)
        return 0

    n2 = E_TILE // 32
    lax.fori_loop(0, n2, body, 0)
    ic = col_v[pl.ds(2 * n2 * 16, 16)]
    plsc.addupdate_scatter(loc_a, [ic], ones)
    plsc.subcore_barrier()

    def red(j, _):
        pltpu.async_copy(loc_a.at[pl.ds(j * 128, 128)],
                         acc.at[idx_v.at[j]], sem_r, add=True)
        pltpu.async_copy(loc_b.at[pl.ds(j * 128, 128)],
                         acc.at[idx_v.at[j]], sem_r, add=True)
        return 0

    lax.fori_loop(0, N_ROWS, red, 0)

    def red_wait(j, _):
        pltpu.make_async_copy(loc_a.at[pl.ds(j * 128, 128)],
                              acc.at[idx_v.at[j]], sem_r).wait()
        pltpu.make_async_copy(loc_b.at[pl.ds(j * 128, 128)],
                              acc.at[idx_v.at[j]], sem_r).wait()
        return 0

    lax.fori_loop(0, N_ROWS, red_wait, 0)
    plsc.subcore_barrier()
    pltpu.sync_copy(acc.at[pl.ds(s * NZ, NZ)], out.at[c].at[pl.ds(s * NZ, NZ)])
    pltpu.sync_copy(acc.at[pl.ds(s * NZ, NZ)], tmp)

    def rep(i, _):
        v16 = tmp[pl.ds(i * 16, 16)]
        for u in range(16):
            vec = jnp.full((16,), v16[u], jnp.float32)
            for k in range(4):
                local_pk[pl.ds((i * 16 + u) * 64 + k * 16, 16)] = vec
        return 0

    lax.fori_loop(0, NZ // 16, rep, 0)
    pltpu.sync_copy(local_pk, out_pk.at[c].at[pl.ds(s * NZ * 64, NZ * 64)])


_DEPTH = 8


@functools.partial(
    pl.kernel,
    mesh=_mesh,
    compiler_params=_sc_params,
    out_type=jax.ShapeDtypeStruct((NC, N_PAD, 64), jnp.float32),
    scratch_types=(
        [pltpu.VMEM((NCHUNK, CHUNK), jnp.int32)] * 2
        + [pltpu.VMEM((CHUNK, 64), jnp.float32)] * _DEPTH
        + [pltpu.VMEM_SHARED((N_PAD, 64), jnp.float32)]
        + [pltpu.SemaphoreType.DMA] * (2 * _DEPTH)
    ),
)
def _sc_agg64(row4, col4, g, zeros64, out, row_v, col_v, *rest):
    bufs = rest[:_DEPTH]
    acc = rest[_DEPTH]
    gsems = rest[_DEPTH + 1:2 * _DEPTH + 1]
    ssems = rest[2 * _DEPTH + 1:]
    c = lax.axis_index("c")
    s = lax.axis_index("s")
    pltpu.sync_copy(row4.at[c].at[s], row_v)
    pltpu.sync_copy(col4.at[c].at[s], col_v)
    nz = N_PAD // NS

    @pl.when(c == 0)
    def _init_g():
        pltpu.sync_copy(g.at[pl.ds(s * nz, nz)], acc.at[pl.ds(s * nz, nz)])

    @pl.when(c != 0)
    def _init_z():
        pltpu.sync_copy(zeros64.at[pl.ds(s * nz, nz)],
                        acc.at[pl.ds(s * nz, nz)])

    plsc.subcore_barrier()

    for k in range(_DEPTH):
        pltpu.async_copy(g.at[row_v.at[k]], bufs[k], gsems[k])

    def body(j, _):
        for k in range(_DEPTH):
            ch = _DEPTH * j + k
            pltpu.make_async_copy(g.at[row_v.at[ch]], bufs[k], gsems[k]).wait()
            pltpu.async_copy(bufs[k], acc.at[col_v.at[ch]], ssems[k],
                             add=True)
        for k in range(_DEPTH):
            nxt = _DEPTH * j + _DEPTH + k
            pltpu.make_async_copy(bufs[k], acc.at[col_v.at[_DEPTH * j + k]],
                                  ssems[k]).wait()
            pltpu.async_copy(g.at[row_v.at[nxt]], bufs[k], gsems[k])
        return 0

    lax.fori_loop(0, NCHUNK // _DEPTH - 1, body, 0)
    for k in range(_DEPTH):
        ch = NCHUNK - _DEPTH + k
        pltpu.make_async_copy(g.at[row_v.at[ch]], bufs[k], gsems[k]).wait()
        pltpu.async_copy(bufs[k], acc.at[col_v.at[ch]], ssems[k], add=True)
    for k in range(_DEPTH):
        pltpu.make_async_copy(bufs[k], acc.at[col_v.at[NCHUNK - _DEPTH + k]],
                              ssems[k]).wait()
    plsc.subcore_barrier()
    pltpu.sync_copy(acc.at[pl.ds(s * nz, nz)], out.at[c].at[pl.ds(s * nz, nz)])


@functools.partial(
    pl.kernel,
    mesh=_mesh,
    compiler_params=_sc_params,
    out_type=[
        jax.ShapeDtypeStruct((NC, N_PAD), jnp.float32),
        jax.ShapeDtypeStruct((N_PAD,), jnp.float32),
    ],
    scratch_types=[
        pltpu.VMEM((E_TILE,), jnp.int32),
        pltpu.VMEM((E_TILE,), jnp.int32),
        pltpu.VMEM((N_PAD,), jnp.float32),
        pltpu.VMEM((N_PAD,), jnp.float32),
        pltpu.VMEM((N_PAD,), jnp.float32),
        pltpu.VMEM((N_ROWS, 128), jnp.int32),
        pltpu.VMEM((N_ROWS, 128), jnp.int32),
        pltpu.VMEM_SHARED((N_PAD,), jnp.float32),
        pltpu.SemaphoreType.DMA,
        pltpu.SemaphoreType.DMA,
    ],
)
def _sc_agg1(rowf, colf, g7f, zeros1d, ident, ident64, out, g7d, row_v,
             col_v, g_local, loc_a, loc_b, idx_v, id64_v, acc, sem_r, sem_g):
    c = lax.axis_index("c")
    s = lax.axis_index("s")
    pltpu.sync_copy(rowf.at[c].at[s], row_v)
    pltpu.sync_copy(colf.at[c].at[s], col_v)
    pltpu.sync_copy(ident, idx_v)
    pltpu.sync_copy(ident64, id64_v)
    pltpu.sync_copy(zeros1d, loc_a)
    pltpu.sync_copy(zeros1d, loc_b)
    pltpu.sync_copy(zeros1d.at[pl.ds(s * NZ, NZ)], acc.at[pl.ds(s * NZ, NZ)])

    def stage(j, _):
        pltpu.async_copy(g7f.at[id64_v.at[j]],
                         g_local.at[pl.ds(j * 128, 128)], sem_g)
        return 0

    lax.fori_loop(0, N_ROWS, stage, 0)

    def stage_wait(j, _):
        pltpu.make_async_copy(g7f.at[id64_v.at[j]],
                              g_local.at[pl.ds(j * 128, 128)], sem_g).wait()
        return 0

    lax.fori_loop(0, N_ROWS, stage_wait, 0)

    @pl.when(c == 0)
    def _g7d():
        pltpu.sync_copy(g_local.at[pl.ds(s * NZ, NZ)],
                        g7d.at[pl.ds(s * NZ, NZ)])

    def body(i, _):
        ir = row_v[pl.ds(2 * i * 16, 16)]
        vals = plsc.load_gather(g_local, [ir])
        ic = col_v[pl.ds(2 * i * 16, 16)]
        plsc.addupdate_scatter(loc_a, [ic], vals)
        ir2 = row_v[pl.ds((2 * i + 1) * 16, 16)]
        vals2 = plsc.load_gather(g_local, [ir2])
        ic2 = col_v[pl.ds((2 * i + 1) * 16, 16)]
        plsc.addupdate_scatter(loc_b, [ic2], vals2)
        return 0

    n2 = E_TILE // 32
    lax.fori_loop(0, n2, body, 0)
    ir = row_v[pl.ds(2 * n2 * 16, 16)]
    vals = plsc.load_gather(g_local, [ir])
    ic = col_v[pl.ds(2 * n2 * 16, 16)]
    plsc.addupdate_scatter(loc_a, [ic], vals)
    plsc.subcore_barrier()

    def red(j, _):
        pltpu.async_copy(loc_a.at[pl.ds(j * 128, 128)],
                         acc.at[idx_v.at[j]], sem_r, add=True)
        pltpu.async_copy(loc_b.at[pl.ds(j * 128, 128)],
                         acc.at[idx_v.at[j]], sem_r, add=True)
        return 0

    lax.fori_loop(0, N_ROWS, red, 0)

    def red_wait(j, _):
        pltpu.make_async_copy(loc_a.at[pl.ds(j * 128, 128)],
                              acc.at[idx_v.at[j]], sem_r).wait()
        pltpu.make_async_copy(loc_b.at[pl.ds(j * 128, 128)],
                              acc.at[idx_v.at[j]], sem_r).wait()
        return 0

    lax.fori_loop(0, N_ROWS, red_wait, 0)
    plsc.subcore_barrier()
    pltpu.sync_copy(acc.at[pl.ds(s * NZ, NZ)], out.at[c].at[pl.ds(s * NZ, NZ)])


_R = 1024


_RP = _R // 2


def _tc_first(x_pk, dpk, W2_i):
    def body(dpk_ref, x_ref, w_ref, dpki_ref, g_ref):
        dinv = lax.rsqrt(1.0 + dpk_ref[0] + dpk_ref[1])
        dpki_ref[...] = dinv
        g_ref[...] = dinv * jnp.dot(x_ref[...], w_ref[...],
                                    preferred_element_type=jnp.float32)

    return pl.pallas_call(
        body,
        grid=(N_PAD // _R,),
        in_specs=[
            pl.BlockSpec((NC, _RP, 128), lambda i: (0, i, 0)),
            pl.BlockSpec((_RP, 256), lambda i: (i, 0)),
            pl.BlockSpec((256, 128), lambda i: (0, 0)),
        ],
        out_specs=[
            pl.BlockSpec((_RP, 128), lambda i: (i, 0)),
            pl.BlockSpec((_RP, 128), lambda i: (i, 0)),
        ],
        out_shape=[
            jax.ShapeDtypeStruct((N_PAD // 2, 128), jnp.float32),
            jax.ShapeDtypeStruct((N_PAD // 2, 128), jnp.float32),
        ],
    )(dpk, x_pk, W2_i)


def _tc_layer_pk(p_pk, dinv_pk, b2, W2):
    def body(p_ref, dinv_ref, b_ref, w_ref, o_ref):
        dinv = dinv_ref[...]
        h = jnp.maximum(dinv * (p_ref[0] + p_ref[1]) + b_ref[...], 0.0)
        o_ref[...] = dinv * jnp.dot(h, w_ref[...],
                                    preferred_element_type=jnp.float32)

    return pl.pallas_call(
        body,
        grid=(N_PAD // _R,),
        in_specs=[
            pl.BlockSpec((NC, _RP, 128), lambda i: (0, i, 0)),
            pl.BlockSpec((_RP, 128), lambda i: (i, 0)),
            pl.BlockSpec((1, 128), lambda i: (0, 0)),
            pl.BlockSpec((128, 128), lambda i: (0, 0)),
        ],
        out_specs=pl.BlockSpec((_RP, 128), lambda i: (i, 0)),
        out_shape=jax.ShapeDtypeStruct((N_PAD // 2, 128), jnp.float32),
    )(p_pk, dinv_pk, b2, W2)


def _tc_final(q, g7d, dn, b_o):
    def body(q_ref, g_ref, d_ref, b_ref, o_ref):
        dinv = lax.rsqrt(1.0 + d_ref[0] + d_ref[1])
        o_ref[...] = dinv * (q_ref[0] + q_ref[1] + g_ref[...]) + b_ref[...]

    return pl.pallas_call(
        body,
        out_shape=jax.ShapeDtypeStruct((N_ROWS, 128), jnp.float32),
    )(q, g7d, dn, b_o)


def kernel(x, edge_index, W_i, b_i, Wh, bh, W_o, b_o):
    row = edge_index[0].astype(jnp.int32)
    col = edge_index[1].astype(jnp.int32)
    row4 = row.reshape(NC, NS, NCHUNK, CHUNK)
    col4 = col.reshape(NC, NS, NCHUNK, CHUNK)
    rowf = row.reshape(NC, NS, E_TILE)
    colf = col.reshape(NC, NS, E_TILE)

    zeros1d = jnp.zeros((N_PAD,), jnp.float32)
    zeros64 = jnp.zeros((N_PAD, 64), jnp.float32)
    ident = jnp.arange(N_PAD, dtype=jnp.int32).reshape(N_ROWS, 128)
    x_pad = jnp.zeros((N_PAD, 128), jnp.float32).at[:N_NODES].set(x)

    def blockdiag2(W):
        din, dout = W.shape
        W2 = jnp.zeros((2 * din, 2 * dout), jnp.float32)
        return W2.at[:din, :dout].set(W).at[din:, dout:].set(W)

    dn_lin, dpk_flat = _sc_degree(colf, zeros1d, ident)
    dpk = dpk_flat.reshape(NC, N_PAD // 2, 128)
    dinv_pk, g = _tc_first(x_pad.reshape(N_PAD // 2, 256), dpk,
                           blockdiag2(W_i))

    for k in range(6):
        b = b_i if k == 0 else bh[k - 1]
        p = _sc_agg64(row4, col4, g.reshape(N_PAD, 64), zeros64)
        g = _tc_layer_pk(p.reshape(NC, N_PAD // 2, 128), dinv_pk,
                         jnp.concatenate([b, b]).reshape(1, 128),
                         blockdiag2(Wh[k]))

    W2_o = jnp.zeros((128, 128), jnp.float32)
    W2_o = W2_o.at[:64, 0].set(W_o[:, 0]).at[64:, 64].set(W_o[:, 0])
    b7 = bh[5]
    p = _sc_agg64(row4, col4, g.reshape(N_PAD, 64), zeros64)
    g7_pk = _tc_layer_pk(p.reshape(NC, N_PAD // 2, 128), dinv_pk,
                         jnp.concatenate([b7, b7]).reshape(1, 128), W2_o)

    ident64 = (jnp.arange(N_PAD, dtype=jnp.int32) * 64).reshape(N_ROWS, 128)
    q, g7d = _sc_agg1(rowf, colf, g7_pk.reshape(N_PAD * 64), zeros1d, ident,
                      ident64)
    out = _tc_final(q.reshape(NC, N_ROWS, 128), g7d.reshape(N_ROWS, 128),
                    dn_lin.reshape(NC, N_ROWS, 128),
                    jnp.broadcast_to(b_o.reshape(1, 1), (N_ROWS, 128)))
    return out.reshape(N_PAD, 1)[:N_NODES]


# speedup vs baseline: 41.5054x; 1.0334x over previous
import functools

import jax
import jax.numpy as jnp
from jax import lax
from jax.experimental import pallas as pl
from jax.experimental.pallas import tpu as pltpu
from jax.experimental.pallas import tpu_sc as plsc

N_NODES = 10000
N_PAD = 10240
N_ROWS = 80
E = 320000
NC = 2
NS = 16
E_SC = E // NC
E_TILE = E_SC // NS
CHUNK = 125
NCHUNK = E_TILE // CHUNK
ROW_SLICE = 8
N_SLICERS = N_ROWS // ROW_SLICE

_mesh = plsc.VectorSubcoreMesh(core_axis_name="c", subcore_axis_name="s")
_sc_params = pltpu.CompilerParams(needs_layout_passes=False,
                                  use_tc_tiling_on_sc=False)


NZ = N_PAD // NS


@functools.partial(
    pl.kernel,
    mesh=_mesh,
    compiler_params=_sc_params,
    out_type=[
        jax.ShapeDtypeStruct((NC, N_PAD), jnp.float32),
        jax.ShapeDtypeStruct((NC, N_PAD * 64), jnp.float32),
    ],
    scratch_types=[
        pltpu.VMEM((E_TILE,), jnp.int32),
        pltpu.VMEM((N_PAD,), jnp.float32),
        pltpu.VMEM((N_PAD,), jnp.float32),
        pltpu.VMEM((N_ROWS, 128), jnp.int32),
        pltpu.VMEM_SHARED((N_PAD,), jnp.float32),
        pltpu.VMEM((NZ,), jnp.float32),
        pltpu.VMEM((NZ * 64,), jnp.float32),
        pltpu.SemaphoreType.DMA,
    ],
)
def _sc_degree(colf, zeros1d, ident, out, out_pk, col_v, loc_a, loc_b, idx_v,
               acc, tmp, local_pk, sem_r):
    c = lax.axis_index("c")
    s = lax.axis_index("s")
    pltpu.sync_copy(colf.at[c].at[s], col_v)
    pltpu.sync_copy(zeros1d, loc_a)
    pltpu.sync_copy(zeros1d, loc_b)
    pltpu.sync_copy(ident, idx_v)
    pltpu.sync_copy(zeros1d.at[pl.ds(s * NZ, NZ)], acc.at[pl.ds(s * NZ, NZ)])
    ones = jnp.full((16,), 1.0, jnp.float32)

    def body(i, _):
        ic = col_v[pl.ds(2 * i * 16, 16)]
        plsc.addupdate_scatter(loc_a, [ic], ones)
        ic2 = col_v[pl.ds((2 * i + 1) * 16, 16)]
        plsc.addupdate_scatter(loc_b, [ic2], ones)
        return 0

    n2 = E_TILE // 32
    lax.fori_loop(0, n2, body, 0)
    ic = col_v[pl.ds(2 * n2 * 16, 16)]
    plsc.addupdate_scatter(loc_a, [ic], ones)
    plsc.subcore_barrier()

    def red(j, _):
        pltpu.async_copy(loc_a.at[pl.ds(j * 128, 128)],
                         acc.at[idx_v.at[j]], sem_r, add=True)
        pltpu.async_copy(loc_b.at[pl.ds(j * 128, 128)],
                         acc.at[idx_v.at[j]], sem_r, add=True)
        return 0

    lax.fori_loop(0, N_ROWS, red, 0)

    def red_wait(j, _):
        pltpu.make_async_copy(loc_a.at[pl.ds(j * 128, 128)],
                              acc.at[idx_v.at[j]], sem_r).wait()
        pltpu.make_async_copy(loc_b.at[pl.ds(j * 128, 128)],
                              acc.at[idx_v.at[j]], sem_r).wait()
        return 0

    lax.fori_loop(0, N_ROWS, red_wait, 0)
    plsc.subcore_barrier()
    pltpu.sync_copy(acc.at[pl.ds(s * NZ, NZ)], out.at[c].at[pl.ds(s * NZ, NZ)])
    pltpu.sync_copy(acc.at[pl.ds(s * NZ, NZ)], tmp)

    def rep(i, _):
        v16 = tmp[pl.ds(i * 16, 16)]
        for u in range(16):
            vec = jnp.full((16,), v16[u], jnp.float32)
            for k in range(4):
                local_pk[pl.ds((i * 16 + u) * 64 + k * 16, 16)] = vec
        return 0

    lax.fori_loop(0, NZ // 16, rep, 0)
    pltpu.sync_copy(local_pk, out_pk.at[c].at[pl.ds(s * NZ * 64, NZ * 64)])


_DEPTH = 8


@functools.partial(
    pl.kernel,
    mesh=_mesh,
    compiler_params=_sc_params,
    out_type=jax.ShapeDtypeStruct((NC, N_PAD, 64), jnp.float32),
    scratch_types=(
        [pltpu.VMEM((NCHUNK, CHUNK), jnp.int32)] * 2
        + [pltpu.VMEM((CHUNK, 64), jnp.float32)] * _DEPTH
        + [pltpu.VMEM_SHARED((N_PAD, 64), jnp.float32)]
        + [pltpu.SemaphoreType.DMA] * (2 * _DEPTH)
    ),
)
def _sc_agg64(row4, col4, g, zeros64, out, row_v, col_v, *rest):
    bufs = rest[:_DEPTH]
    acc = rest[_DEPTH]
    gsems = rest[_DEPTH + 1:2 * _DEPTH + 1]
    ssems = rest[2 * _DEPTH + 1:]
    c = lax.axis_index("c")
    s = lax.axis_index("s")
    pltpu.sync_copy(row4.at[c].at[s], row_v)
    pltpu.sync_copy(col4.at[c].at[s], col_v)
    nz = N_PAD // NS

    @pl.when(c == 0)
    def _init_g():
        pltpu.sync_copy(g.at[pl.ds(s * nz, nz)], acc.at[pl.ds(s * nz, nz)])

    @pl.when(c != 0)
    def _init_z():
        pltpu.sync_copy(zeros64.at[pl.ds(s * nz, nz)],
                        acc.at[pl.ds(s * nz, nz)])

    plsc.subcore_barrier()

    for k in range(_DEPTH):
        pltpu.async_copy(g.at[row_v.at[k]], bufs[k], gsems[k])

    def body(j, _):
        for k in range(_DEPTH):
            ch = _DEPTH * j + k
            pltpu.make_async_copy(g.at[row_v.at[ch]], bufs[k], gsems[k]).wait()
            pltpu.async_copy(bufs[k], acc.at[col_v.at[ch]], ssems[k],
                             add=True)
        for k in range(_DEPTH):
            nxt = _DEPTH * j + _DEPTH + k
            pltpu.make_async_copy(bufs[k], acc.at[col_v.at[_DEPTH * j + k]],
                                  ssems[k]).wait()
            pltpu.async_copy(g.at[row_v.at[nxt]], bufs[k], gsems[k])
        return 0

    lax.fori_loop(0, NCHUNK // _DEPTH - 1, body, 0)
    for k in range(_DEPTH):
        ch = NCHUNK - _DEPTH + k
        pltpu.make_async_copy(g.at[row_v.at[ch]], bufs[k], gsems[k]).wait()
        pltpu.async_copy(bufs[k], acc.at[col_v.at[ch]], ssems[k], add=True)
    for k in range(_DEPTH):
        pltpu.make_async_copy(bufs[k], acc.at[col_v.at[NCHUNK - _DEPTH + k]],
                              ssems[k]).wait()
    plsc.subcore_barrier()
    pltpu.sync_copy(acc.at[pl.ds(s * nz, nz)], out.at[c].at[pl.ds(s * nz, nz)])


@functools.partial(
    pl.kernel,
    mesh=_mesh,
    compiler_params=_sc_params,
    out_type=[
        jax.ShapeDtypeStruct((NC, N_PAD), jnp.float32),
        jax.ShapeDtypeStruct((N_PAD,), jnp.float32),
    ],
    scratch_types=[
        pltpu.VMEM((E_TILE,), jnp.int32),
        pltpu.VMEM((E_TILE,), jnp.int32),
        pltpu.VMEM((N_PAD,), jnp.float32),
        pltpu.VMEM((N_PAD,), jnp.float32),
        pltpu.VMEM((N_PAD,), jnp.float32),
        pltpu.VMEM((N_ROWS, 128), jnp.int32),
        pltpu.VMEM((N_ROWS, 128), jnp.int32),
        pltpu.VMEM_SHARED((N_PAD,), jnp.float32),
        pltpu.SemaphoreType.DMA,
        pltpu.SemaphoreType.DMA,
    ],
)
def _sc_agg1(rowf, colf, g7f, zeros1d, ident, ident64, out, g7d, row_v,
             col_v, g_local, loc_a, loc_b, idx_v, id64_v, acc, sem_r, sem_g):
    c = lax.axis_index("c")
    s = lax.axis_index("s")
    pltpu.sync_copy(rowf.at[c].at[s], row_v)
    pltpu.sync_copy(colf.at[c].at[s], col_v)
    pltpu.sync_copy(ident, idx_v)
    pltpu.sync_copy(ident64, id64_v)
    pltpu.sync_copy(zeros1d, loc_a)
    pltpu.sync_copy(zeros1d, loc_b)
    pltpu.sync_copy(zeros1d.at[pl.ds(s * NZ, NZ)], acc.at[pl.ds(s * NZ, NZ)])

    def stage(j, _):
        pltpu.async_copy(g7f.at[id64_v.at[j]],
                         g_local.at[pl.ds(j * 128, 128)], sem_g)
        return 0

    lax.fori_loop(0, N_ROWS, stage, 0)

    def stage_wait(j, _):
        pltpu.make_async_copy(g7f.at[id64_v.at[j]],
                              g_local.at[pl.ds(j * 128, 128)], sem_g).wait()
        return 0

    lax.fori_loop(0, N_ROWS, stage_wait, 0)

    @pl.when(c == 0)
    def _g7d():
        pltpu.sync_copy(g_local.at[pl.ds(s * NZ, NZ)],
                        g7d.at[pl.ds(s * NZ, NZ)])

    def body(i, _):
        ir = row_v[pl.ds(2 * i * 16, 16)]
        vals = plsc.load_gather(g_local, [ir])
        ic = col_v[pl.ds(2 * i * 16, 16)]
        plsc.addupdate_scatter(loc_a, [ic], vals)
        ir2 = row_v[pl.ds((2 * i + 1) * 16, 16)]
        vals2 = plsc.load_gather(g_local, [ir2])
        ic2 = col_v[pl.ds((2 * i + 1) * 16, 16)]
        plsc.addupdate_scatter(loc_b, [ic2], vals2)
        return 0

    n2 = E_TILE // 32
    lax.fori_loop(0, n2, body, 0)
    ir = row_v[pl.ds(2 * n2 * 16, 16)]
    vals = plsc.load_gather(g_local, [ir])
    ic = col_v[pl.ds(2 * n2 * 16, 16)]
    plsc.addupdate_scatter(loc_a, [ic], vals)
    plsc.subcore_barrier()

    def red(j, _):
        pltpu.async_copy(loc_a.at[pl.ds(j * 128, 128)],
                         acc.at[idx_v.at[j]], sem_r, add=True)
        pltpu.async_copy(loc_b.at[pl.ds(j * 128, 128)],
                         acc.at[idx_v.at[j]], sem_r, add=True)
        return 0

    lax.fori_loop(0, N_ROWS, red, 0)

    def red_wait(j, _):
        pltpu.make_async_copy(loc_a.at[pl.ds(j * 128, 128)],
                              acc.at[idx_v.at[j]], sem_r).wait()
        pltpu.make_async_copy(loc_b.at[pl.ds(j * 128, 128)],
                              acc.at[idx_v.at[j]], sem_r).wait()
        return 0

    lax.fori_loop(0, N_ROWS, red_wait, 0)
    plsc.subcore_barrier()
    pltpu.sync_copy(acc.at[pl.ds(s * NZ, NZ)], out.at[c].at[pl.ds(s * NZ, NZ)])


_R = 2048


_RP = _R // 2


def _tc_first(x_pk, dpk, W2_i):
    def body(dpk_ref, x_ref, w_ref, dpki_ref, g_ref):
        dinv = lax.rsqrt(1.0 + dpk_ref[0] + dpk_ref[1])
        dpki_ref[...] = dinv
        g_ref[...] = dinv * jnp.dot(x_ref[...], w_ref[...],
                                    preferred_element_type=jnp.float32)

    return pl.pallas_call(
        body,
        grid=(N_PAD // _R,),
        in_specs=[
            pl.BlockSpec((NC, _RP, 128), lambda i: (0, i, 0)),
            pl.BlockSpec((_RP, 256), lambda i: (i, 0)),
            pl.BlockSpec((256, 128), lambda i: (0, 0)),
        ],
        out_specs=[
            pl.BlockSpec((_RP, 128), lambda i: (i, 0)),
            pl.BlockSpec((_RP, 128), lambda i: (i, 0)),
        ],
        out_shape=[
            jax.ShapeDtypeStruct((N_PAD // 2, 128), jnp.float32),
            jax.ShapeDtypeStruct((N_PAD // 2, 128), jnp.float32),
        ],
    )(dpk, x_pk, W2_i)


def _tc_layer_pk(p_pk, dinv_pk, b2, W2):
    def body(p_ref, dinv_ref, b_ref, w_ref, o_ref):
        dinv = dinv_ref[...]
        h = jnp.maximum(dinv * (p_ref[0] + p_ref[1]) + b_ref[...], 0.0)
        o_ref[...] = dinv * jnp.dot(h, w_ref[...],
                                    preferred_element_type=jnp.float32)

    return pl.pallas_call(
        body,
        grid=(N_PAD // _R,),
        in_specs=[
            pl.BlockSpec((NC, _RP, 128), lambda i: (0, i, 0)),
            pl.BlockSpec((_RP, 128), lambda i: (i, 0)),
            pl.BlockSpec((1, 128), lambda i: (0, 0)),
            pl.BlockSpec((128, 128), lambda i: (0, 0)),
        ],
        out_specs=pl.BlockSpec((_RP, 128), lambda i: (i, 0)),
        out_shape=jax.ShapeDtypeStruct((N_PAD // 2, 128), jnp.float32),
    )(p_pk, dinv_pk, b2, W2)


def _tc_final(q, g7d, dn, b_o):
    def body(q_ref, g_ref, d_ref, b_ref, o_ref):
        dinv = lax.rsqrt(1.0 + d_ref[0] + d_ref[1])
        o_ref[...] = dinv * (q_ref[0] + q_ref[1] + g_ref[...]) + b_ref[...]

    return pl.pallas_call(
        body,
        out_shape=jax.ShapeDtypeStruct((N_ROWS, 128), jnp.float32),
    )(q, g7d, dn, b_o)


def kernel(x, edge_index, W_i, b_i, Wh, bh, W_o, b_o):
    row = edge_index[0].astype(jnp.int32)
    col = edge_index[1].astype(jnp.int32)
    row4 = row.reshape(NC, NS, NCHUNK, CHUNK)
    col4 = col.reshape(NC, NS, NCHUNK, CHUNK)
    rowf = row.reshape(NC, NS, E_TILE)
    colf = col.reshape(NC, NS, E_TILE)

    zeros1d = jnp.zeros((N_PAD,), jnp.float32)
    zeros64 = jnp.zeros((N_PAD, 64), jnp.float32)
    ident = jnp.arange(N_PAD, dtype=jnp.int32).reshape(N_ROWS, 128)
    x_pad = jnp.zeros((N_PAD, 128), jnp.float32).at[:N_NODES].set(x)

    def blockdiag2(W):
        din, dout = W.shape
        W2 = jnp.zeros((2 * din, 2 * dout), jnp.float32)
        return W2.at[:din, :dout].set(W).at[din:, dout:].set(W)

    dn_lin, dpk_flat = _sc_degree(colf, zeros1d, ident)
    dpk = dpk_flat.reshape(NC, N_PAD // 2, 128)
    dinv_pk, g = _tc_first(x_pad.reshape(N_PAD // 2, 256), dpk,
                           blockdiag2(W_i))

    for k in range(6):
        b = b_i if k == 0 else bh[k - 1]
        p = _sc_agg64(row4, col4, g.reshape(N_PAD, 64), zeros64)
        g = _tc_layer_pk(p.reshape(NC, N_PAD // 2, 128), dinv_pk,
                         jnp.concatenate([b, b]).reshape(1, 128),
                         blockdiag2(Wh[k]))

    W2_o = jnp.zeros((128, 128), jnp.float32)
    W2_o = W2_o.at[:64, 0].set(W_o[:, 0]).at[64:, 64].set(W_o[:, 0])
    b7 = bh[5]
    p = _sc_agg64(row4, col4, g.reshape(N_PAD, 64), zeros64)
    g7_pk = _tc_layer_pk(p.reshape(NC, N_PAD // 2, 128), dinv_pk,
                         jnp.concatenate([b7, b7]).reshape(1, 128), W2_o)

    ident64 = (jnp.arange(N_PAD, dtype=jnp.int32) * 64).reshape(N_ROWS, 128)
    q, g7d = _sc_agg1(rowf, colf, g7_pk.reshape(N_PAD * 64), zeros1d, ident,
                      ident64)
    out = _tc_final(q.reshape(NC, N_ROWS, 128), g7d.reshape(N_ROWS, 128),
                    dn_lin.reshape(NC, N_ROWS, 128),
                    jnp.broadcast_to(b_o.reshape(1, 1), (N_ROWS, 128)))
    return out.reshape(N_PAD, 1)[:N_NODES]


# speedup vs baseline: 42.0714x; 1.0136x over previous
import functools

import jax
import jax.numpy as jnp
from jax import lax
from jax.experimental import pallas as pl
from jax.experimental.pallas import tpu as pltpu
from jax.experimental.pallas import tpu_sc as plsc

N_NODES = 10000
N_PAD = 10240
N_ROWS = 80
E = 320000
NC = 2
NS = 16
E_SC = E // NC
E_TILE = E_SC // NS
CHUNK = 125
NCHUNK = E_TILE // CHUNK
ROW_SLICE = 8
N_SLICERS = N_ROWS // ROW_SLICE

_mesh = plsc.VectorSubcoreMesh(core_axis_name="c", subcore_axis_name="s")
_sc_params = pltpu.CompilerParams(needs_layout_passes=False,
                                  use_tc_tiling_on_sc=False)


NZ = N_PAD // NS


@functools.partial(
    pl.kernel,
    mesh=_mesh,
    compiler_params=_sc_params,
    out_type=[
        jax.ShapeDtypeStruct((NC, N_PAD), jnp.float32),
        jax.ShapeDtypeStruct((NC, N_PAD * 64), jnp.float32),
    ],
    scratch_types=[
        pltpu.VMEM((E_TILE,), jnp.int32),
        pltpu.VMEM((N_PAD,), jnp.float32),
        pltpu.VMEM((N_PAD,), jnp.float32),
        pltpu.VMEM((N_ROWS, 128), jnp.int32),
        pltpu.VMEM_SHARED((N_PAD,), jnp.float32),
        pltpu.VMEM((NZ,), jnp.float32),
        pltpu.VMEM((NZ * 64,), jnp.float32),
        pltpu.SemaphoreType.DMA,
    ],
)
def _sc_degree(colf, zeros1d, ident, out, out_pk, col_v, loc_a, loc_b, idx_v,
               acc, tmp, local_pk, sem_r):
    c = lax.axis_index("c")
    s = lax.axis_index("s")
    pltpu.sync_copy(colf.at[c].at[s], col_v)
    pltpu.sync_copy(zeros1d, loc_a)
    pltpu.sync_copy(zeros1d, loc_b)
    pltpu.sync_copy(ident, idx_v)
    pltpu.sync_copy(zeros1d.at[pl.ds(s * NZ, NZ)], acc.at[pl.ds(s * NZ, NZ)])
    ones = jnp.full((16,), 1.0, jnp.float32)

    def body(i, _):
        ic = col_v[pl.ds(2 * i * 16, 16)]
        plsc.addupdate_scatter(loc_a, [ic], ones)
        ic2 = col_v[pl.ds((2 * i + 1) * 16, 16)]
        plsc.addupdate_scatter(loc_b, [ic2], ones)
        return 0

    n2 = E_TILE // 32
    lax.fori_loop(0, n2, body, 0)
    ic = col_v[pl.ds(2 * n2 * 16, 16)]
    plsc.addupdate_scatter(loc_a, [ic], ones)
    plsc.subcore_barrier()

    def red(j, _):
        pltpu.async_copy(loc_a.at[pl.ds(j * 128, 128)],
                         acc.at[idx_v.at[j]], sem_r, add=True)
        pltpu.async_copy(loc_b.at[pl.ds(j * 128, 128)],
                         acc.at[idx_v.at[j]], sem_r, add=True)
        return 0

    lax.fori_loop(0, N_ROWS, red, 0)

    def red_wait(j, _):
        pltpu.make_async_copy(loc_a.at[pl.ds(j * 128, 128)],
                              acc.at[idx_v.at[j]], sem_r).wait()
        pltpu.make_async_copy(loc_b.at[pl.ds(j * 128, 128)],
                              acc.at[idx_v.at[j]], sem_r).wait()
        return 0

    lax.fori_loop(0, N_ROWS, red_wait, 0)
    plsc.subcore_barrier()
    pltpu.sync_copy(acc.at[pl.ds(s * NZ, NZ)], out.at[c].at[pl.ds(s * NZ, NZ)])
    pltpu.sync_copy(acc.at[pl.ds(s * NZ, NZ)], tmp)

    def rep(i, _):
        v16 = tmp[pl.ds(i * 16, 16)]
        for u in range(16):
            vec = jnp.full((16,), v16[u], jnp.float32)
            for k in range(4):
                local_pk[pl.ds((i * 16 + u) * 64 + k * 16, 16)] = vec
        return 0

    lax.fori_loop(0, NZ // 16, rep, 0)
    pltpu.sync_copy(local_pk, out_pk.at[c].at[pl.ds(s * NZ * 64, NZ * 64)])


_DEPTH = 8


@functools.partial(
    pl.kernel,
    mesh=_mesh,
    compiler_params=_sc_params,
    out_type=jax.ShapeDtypeStruct((NC, N_PAD, 64), jnp.float32),
    scratch_types=(
        [pltpu.VMEM((NCHUNK, CHUNK), jnp.int32)] * 2
        + [pltpu.VMEM((CHUNK, 64), jnp.float32)] * _DEPTH
        + [pltpu.VMEM_SHARED((N_PAD, 64), jnp.float32)]
        + [pltpu.SemaphoreType.DMA] * (2 * _DEPTH)
    ),
)
def _sc_agg64(row4, col4, g, zeros64, out, row_v, col_v, *rest):
    bufs = rest[:_DEPTH]
    acc = rest[_DEPTH]
    gsems = rest[_DEPTH + 1:2 * _DEPTH + 1]
    ssems = rest[2 * _DEPTH + 1:]
    c = lax.axis_index("c")
    s = lax.axis_index("s")
    pltpu.sync_copy(row4.at[c].at[s], row_v)
    pltpu.sync_copy(col4.at[c].at[s], col_v)
    nz = N_PAD // NS

    @pl.when(c == 0)
    def _init_g():
        pltpu.sync_copy(g.at[pl.ds(s * nz, nz)], acc.at[pl.ds(s * nz, nz)])

    @pl.when(c != 0)
    def _init_z():
        pltpu.sync_copy(zeros64.at[pl.ds(s * nz, nz)],
                        acc.at[pl.ds(s * nz, nz)])

    plsc.subcore_barrier()

    for k in range(_DEPTH):
        pltpu.async_copy(g.at[row_v.at[k]], bufs[k], gsems[k])

    def body(j, _):
        for k in range(_DEPTH):
            ch = _DEPTH * j + k
            pltpu.make_async_copy(g.at[row_v.at[ch]], bufs[k], gsems[k]).wait()
            pltpu.async_copy(bufs[k], acc.at[col_v.at[ch]], ssems[k],
                             add=True)
        for k in range(_DEPTH):
            nxt = _DEPTH * j + _DEPTH + k
            pltpu.make_async_copy(bufs[k], acc.at[col_v.at[_DEPTH * j + k]],
                                  ssems[k]).wait()
            pltpu.async_copy(g.at[row_v.at[nxt]], bufs[k], gsems[k])
        return 0

    lax.fori_loop(0, NCHUNK // _DEPTH - 1, body, 0)
    for k in range(_DEPTH):
        ch = NCHUNK - _DEPTH + k
        pltpu.make_async_copy(g.at[row_v.at[ch]], bufs[k], gsems[k]).wait()
        pltpu.async_copy(bufs[k], acc.at[col_v.at[ch]], ssems[k], add=True)
    for k in range(_DEPTH):
        pltpu.make_async_copy(bufs[k], acc.at[col_v.at[NCHUNK - _DEPTH + k]],
                              ssems[k]).wait()
    plsc.subcore_barrier()
    pltpu.sync_copy(acc.at[pl.ds(s * nz, nz)], out.at[c].at[pl.ds(s * nz, nz)])


@functools.partial(
    pl.kernel,
    mesh=_mesh,
    compiler_params=_sc_params,
    out_type=[
        jax.ShapeDtypeStruct((NC, N_PAD), jnp.float32),
        jax.ShapeDtypeStruct((N_PAD,), jnp.float32),
    ],
    scratch_types=[
        pltpu.VMEM((E_TILE,), jnp.int32),
        pltpu.VMEM((E_TILE,), jnp.int32),
        pltpu.VMEM((N_PAD,), jnp.float32),
        pltpu.VMEM((N_PAD,), jnp.float32),
        pltpu.VMEM((N_PAD,), jnp.float32),
        pltpu.VMEM((N_ROWS, 128), jnp.int32),
        pltpu.VMEM((N_ROWS, 128), jnp.int32),
        pltpu.VMEM_SHARED((N_PAD,), jnp.float32),
        pltpu.SemaphoreType.DMA,
        pltpu.SemaphoreType.DMA,
    ],
)
def _sc_agg1(rowf, colf, g7f, zeros1d, ident, ident64, out, g7d, row_v,
             col_v, g_local, loc_a, loc_b, idx_v, id64_v, acc, sem_r, sem_g):
    c = lax.axis_index("c")
    s = lax.axis_index("s")
    pltpu.sync_copy(rowf.at[c].at[s], row_v)
    pltpu.sync_copy(colf.at[c].at[s], col_v)
    pltpu.sync_copy(ident, idx_v)
    pltpu.sync_copy(ident64, id64_v)
    pltpu.sync_copy(zeros1d, loc_a)
    pltpu.sync_copy(zeros1d, loc_b)
    pltpu.sync_copy(zeros1d.at[pl.ds(s * NZ, NZ)], acc.at[pl.ds(s * NZ, NZ)])

    def stage(j, _):
        pltpu.async_copy(g7f.at[id64_v.at[j]],
                         g_local.at[pl.ds(j * 128, 128)], sem_g)
        return 0

    lax.fori_loop(0, N_ROWS, stage, 0)

    def stage_wait(j, _):
        pltpu.make_async_copy(g7f.at[id64_v.at[j]],
                              g_local.at[pl.ds(j * 128, 128)], sem_g).wait()
        return 0

    lax.fori_loop(0, N_ROWS, stage_wait, 0)

    @pl.when(c == 0)
    def _g7d():
        pltpu.sync_copy(g_local.at[pl.ds(s * NZ, NZ)],
                        g7d.at[pl.ds(s * NZ, NZ)])

    def body(i, _):
        ir = row_v[pl.ds(2 * i * 16, 16)]
        vals = plsc.load_gather(g_local, [ir])
        ic = col_v[pl.ds(2 * i * 16, 16)]
        plsc.addupdate_scatter(loc_a, [ic], vals)
        ir2 = row_v[pl.ds((2 * i + 1) * 16, 16)]
        vals2 = plsc.load_gather(g_local, [ir2])
        ic2 = col_v[pl.ds((2 * i + 1) * 16, 16)]
        plsc.addupdate_scatter(loc_b, [ic2], vals2)
        return 0

    n2 = E_TILE // 32
    lax.fori_loop(0, n2, body, 0)
    ir = row_v[pl.ds(2 * n2 * 16, 16)]
    vals = plsc.load_gather(g_local, [ir])
    ic = col_v[pl.ds(2 * n2 * 16, 16)]
    plsc.addupdate_scatter(loc_a, [ic], vals)
    plsc.subcore_barrier()

    def red(j, _):
        pltpu.async_copy(loc_a.at[pl.ds(j * 128, 128)],
                         acc.at[idx_v.at[j]], sem_r, add=True)
        pltpu.async_copy(loc_b.at[pl.ds(j * 128, 128)],
                         acc.at[idx_v.at[j]], sem_r, add=True)
        return 0

    lax.fori_loop(0, N_ROWS, red, 0)

    def red_wait(j, _):
        pltpu.make_async_copy(loc_a.at[pl.ds(j * 128, 128)],
                              acc.at[idx_v.at[j]], sem_r).wait()
        pltpu.make_async_copy(loc_b.at[pl.ds(j * 128, 128)],
                              acc.at[idx_v.at[j]], sem_r).wait()
        return 0

    lax.fori_loop(0, N_ROWS, red_wait, 0)
    plsc.subcore_barrier()
    pltpu.sync_copy(acc.at[pl.ds(s * NZ, NZ)], out.at[c].at[pl.ds(s * NZ, NZ)])


_R = 2560


_RP = _R // 2


def _tc_first(x_pk, dpk, W2_i):
    def body(dpk_ref, x_ref, w_ref, dpki_ref, g_ref):
        dinv = lax.rsqrt(1.0 + dpk_ref[0] + dpk_ref[1])
        dpki_ref[...] = dinv
        g_ref[...] = dinv * jnp.dot(x_ref[...], w_ref[...],
                                    preferred_element_type=jnp.float32)

    return pl.pallas_call(
        body,
        grid=(N_PAD // _R,),
        in_specs=[
            pl.BlockSpec((NC, _RP, 128), lambda i: (0, i, 0)),
            pl.BlockSpec((_RP, 256), lambda i: (i, 0)),
            pl.BlockSpec((256, 128), lambda i: (0, 0)),
        ],
        out_specs=[
            pl.BlockSpec((_RP, 128), lambda i: (i, 0)),
            pl.BlockSpec((_RP, 128), lambda i: (i, 0)),
        ],
        out_shape=[
            jax.ShapeDtypeStruct((N_PAD // 2, 128), jnp.float32),
            jax.ShapeDtypeStruct((N_PAD // 2, 128), jnp.float32),
        ],
    )(dpk, x_pk, W2_i)


def _tc_layer_pk(p_pk, dinv_pk, b2, W2):
    def body(p_ref, dinv_ref, b_ref, w_ref, o_ref):
        dinv = dinv_ref[...]
        h = jnp.maximum(dinv * (p_ref[0] + p_ref[1]) + b_ref[...], 0.0)
        o_ref[...] = dinv * jnp.dot(h, w_ref[...],
                                    preferred_element_type=jnp.float32)

    return pl.pallas_call(
        body,
        grid=(N_PAD // _R,),
        in_specs=[
            pl.BlockSpec((NC, _RP, 128), lambda i: (0, i, 0)),
            pl.BlockSpec((_RP, 128), lambda i: (i, 0)),
            pl.BlockSpec((1, 128), lambda i: (0, 0)),
            pl.BlockSpec((128, 128), lambda i: (0, 0)),
        ],
        out_specs=pl.BlockSpec((_RP, 128), lambda i: (i, 0)),
        out_shape=jax.ShapeDtypeStruct((N_PAD // 2, 128), jnp.float32),
    )(p_pk, dinv_pk, b2, W2)


def _tc_final(q, g7d, dn, b_o):
    def body(q_ref, g_ref, d_ref, b_ref, o_ref):
        dinv = lax.rsqrt(1.0 + d_ref[0] + d_ref[1])
        o_ref[...] = dinv * (q_ref[0] + q_ref[1] + g_ref[...]) + b_ref[...]

    return pl.pallas_call(
        body,
        out_shape=jax.ShapeDtypeStruct((N_ROWS, 128), jnp.float32),
    )(q, g7d, dn, b_o)


def kernel(x, edge_index, W_i, b_i, Wh, bh, W_o, b_o):
    row = edge_index[0].astype(jnp.int32)
    col = edge_index[1].astype(jnp.int32)
    row4 = row.reshape(NC, NS, NCHUNK, CHUNK)
    col4 = col.reshape(NC, NS, NCHUNK, CHUNK)
    rowf = row.reshape(NC, NS, E_TILE)
    colf = col.reshape(NC, NS, E_TILE)

    zeros1d = jnp.zeros((N_PAD,), jnp.float32)
    zeros64 = jnp.zeros((N_PAD, 64), jnp.float32)
    ident = jnp.arange(N_PAD, dtype=jnp.int32).reshape(N_ROWS, 128)
    x_pad = jnp.zeros((N_PAD, 128), jnp.float32).at[:N_NODES].set(x)

    def blockdiag2(W):
        din, dout = W.shape
        W2 = jnp.zeros((2 * din, 2 * dout), jnp.float32)
        return W2.at[:din, :dout].set(W).at[din:, dout:].set(W)

    dn_lin, dpk_flat = _sc_degree(colf, zeros1d, ident)
    dpk = dpk_flat.reshape(NC, N_PAD // 2, 128)
    dinv_pk, g = _tc_first(x_pad.reshape(N_PAD // 2, 256), dpk,
                           blockdiag2(W_i))

    for k in range(6):
        b = b_i if k == 0 else bh[k - 1]
        p = _sc_agg64(row4, col4, g.reshape(N_PAD, 64), zeros64)
        g = _tc_layer_pk(p.reshape(NC, N_PAD // 2, 128), dinv_pk,
                         jnp.concatenate([b, b]).reshape(1, 128),
                         blockdiag2(Wh[k]))

    W2_o = jnp.zeros((128, 128), jnp.float32)
    W2_o = W2_o.at[:64, 0].set(W_o[:, 0]).at[64:, 64].set(W_o[:, 0])
    b7 = bh[5]
    p = _sc_agg64(row4, col4, g.reshape(N_PAD, 64), zeros64)
    g7_pk = _tc_layer_pk(p.reshape(NC, N_PAD // 2, 128), dinv_pk,
                         jnp.concatenate([b7, b7]).reshape(1, 128), W2_o)

    ident64 = (jnp.arange(N_PAD, dtype=jnp.int32) * 64).reshape(N_ROWS, 128)
    q, g7d = _sc_agg1(rowf, colf, g7_pk.reshape(N_PAD * 64), zeros1d, ident,
                      ident64)
    out = _tc_final(q.reshape(NC, N_ROWS, 128), g7d.reshape(N_ROWS, 128),
                    dn_lin.reshape(NC, N_ROWS, 128),
                    jnp.broadcast_to(b_o.reshape(1, 1), (N_ROWS, 128)))
    return out.reshape(N_PAD, 1)[:N_NODES]


# speedup vs baseline: 42.7836x; 1.0169x over previous
import functools

import jax
import jax.numpy as jnp
from jax import lax
from jax.experimental import pallas as pl
from jax.experimental.pallas import tpu as pltpu
from jax.experimental.pallas import tpu_sc as plsc

N_NODES = 10000
N_PAD = 10240
N_ROWS = 80
E = 320000
NC = 2
NS = 16
E_SC = E // NC
E_TILE = E_SC // NS
CHUNK = 125
NCHUNK = E_TILE // CHUNK
ROW_SLICE = 8
N_SLICERS = N_ROWS // ROW_SLICE

_mesh = plsc.VectorSubcoreMesh(core_axis_name="c", subcore_axis_name="s")
_sc_params = pltpu.CompilerParams(needs_layout_passes=False,
                                  use_tc_tiling_on_sc=False)


NZ = N_PAD // NS


@functools.partial(
    pl.kernel,
    mesh=_mesh,
    compiler_params=_sc_params,
    out_type=[
        jax.ShapeDtypeStruct((NC, N_PAD), jnp.float32),
        jax.ShapeDtypeStruct((NC, N_PAD * 64), jnp.float32),
    ],
    scratch_types=[
        pltpu.VMEM((E_TILE,), jnp.int32),
        pltpu.VMEM((N_PAD,), jnp.float32),
        pltpu.VMEM((N_PAD,), jnp.float32),
        pltpu.VMEM((N_ROWS, 128), jnp.int32),
        pltpu.VMEM_SHARED((N_PAD,), jnp.float32),
        pltpu.VMEM((NZ,), jnp.float32),
        pltpu.VMEM((NZ * 64,), jnp.float32),
        pltpu.SemaphoreType.DMA,
    ],
)
def _sc_degree(colf, zeros1d, ident, out, out_pk, col_v, loc_a, loc_b, idx_v,
               acc, tmp, local_pk, sem_r):
    c = lax.axis_index("c")
    s = lax.axis_index("s")
    pltpu.sync_copy(colf.at[c].at[s], col_v)
    pltpu.sync_copy(zeros1d, loc_a)
    pltpu.sync_copy(zeros1d, loc_b)
    pltpu.sync_copy(ident, idx_v)
    pltpu.sync_copy(zeros1d.at[pl.ds(s * NZ, NZ)], acc.at[pl.ds(s * NZ, NZ)])
    ones = jnp.full((16,), 1.0, jnp.float32)

    def body(i, _):
        ic = col_v[pl.ds(2 * i * 16, 16)]
        plsc.addupdate_scatter(loc_a, [ic], ones)
        ic2 = col_v[pl.ds((2 * i + 1) * 16, 16)]
        plsc.addupdate_scatter(loc_b, [ic2], ones)
        return 0

    n2 = E_TILE // 32
    lax.fori_loop(0, n2, body, 0)
    ic = col_v[pl.ds(2 * n2 * 16, 16)]
    plsc.addupdate_scatter(loc_a, [ic], ones)
    plsc.subcore_barrier()

    def red(j, _):
        pltpu.async_copy(loc_a.at[pl.ds(j * 128, 128)],
                         acc.at[idx_v.at[j]], sem_r, add=True)
        pltpu.async_copy(loc_b.at[pl.ds(j * 128, 128)],
                         acc.at[idx_v.at[j]], sem_r, add=True)
        return 0

    lax.fori_loop(0, N_ROWS, red, 0)

    def red_wait(j, _):
        pltpu.make_async_copy(loc_a.at[pl.ds(j * 128, 128)],
                              acc.at[idx_v.at[j]], sem_r).wait()
        pltpu.make_async_copy(loc_b.at[pl.ds(j * 128, 128)],
                              acc.at[idx_v.at[j]], sem_r).wait()
        return 0

    lax.fori_loop(0, N_ROWS, red_wait, 0)
    plsc.subcore_barrier()
    pltpu.sync_copy(acc.at[pl.ds(s * NZ, NZ)], out.at[c].at[pl.ds(s * NZ, NZ)])
    pltpu.sync_copy(acc.at[pl.ds(s * NZ, NZ)], tmp)

    def rep(i, _):
        v16 = tmp[pl.ds(i * 16, 16)]
        for u in range(16):
            vec = jnp.full((16,), v16[u], jnp.float32)
            for k in range(4):
                local_pk[pl.ds((i * 16 + u) * 64 + k * 16, 16)] = vec
        return 0

    lax.fori_loop(0, NZ // 16, rep, 0)
    pltpu.sync_copy(local_pk, out_pk.at[c].at[pl.ds(s * NZ * 64, NZ * 64)])


_DEPTH = 8


@functools.partial(
    pl.kernel,
    mesh=_mesh,
    compiler_params=_sc_params,
    out_type=jax.ShapeDtypeStruct((NC, N_PAD, 64), jnp.float32),
    scratch_types=(
        [pltpu.VMEM((NCHUNK, CHUNK), jnp.int32)] * 2
        + [pltpu.VMEM((CHUNK, 64), jnp.float32)] * _DEPTH
        + [pltpu.VMEM_SHARED((N_PAD, 64), jnp.float32)]
        + [pltpu.SemaphoreType.DMA] * (2 * _DEPTH)
    ),
)
def _sc_agg64(row4, col4, g, zeros64, out, row_v, col_v, *rest):
    bufs = rest[:_DEPTH]
    acc = rest[_DEPTH]
    gsems = rest[_DEPTH + 1:2 * _DEPTH + 1]
    ssems = rest[2 * _DEPTH + 1:]
    c = lax.axis_index("c")
    s = lax.axis_index("s")
    pltpu.sync_copy(row4.at[c].at[s], row_v)
    pltpu.sync_copy(col4.at[c].at[s], col_v)
    nz = N_PAD // NS

    @pl.when(c == 0)
    def _init_g():
        pltpu.sync_copy(g.at[pl.ds(s * nz, nz)], acc.at[pl.ds(s * nz, nz)])

    @pl.when(c != 0)
    def _init_z():
        pltpu.sync_copy(zeros64.at[pl.ds(s * nz, nz)],
                        acc.at[pl.ds(s * nz, nz)])

    plsc.subcore_barrier()

    for k in range(_DEPTH):
        pltpu.async_copy(g.at[row_v.at[k]], bufs[k], gsems[k])

    def body(j, _):
        for k in range(_DEPTH):
            ch = _DEPTH * j + k
            pltpu.make_async_copy(g.at[row_v.at[ch]], bufs[k], gsems[k]).wait()
            pltpu.async_copy(bufs[k], acc.at[col_v.at[ch]], ssems[k],
                             add=True)
        for k in range(_DEPTH):
            nxt = _DEPTH * j + _DEPTH + k
            pltpu.make_async_copy(bufs[k], acc.at[col_v.at[_DEPTH * j + k]],
                                  ssems[k]).wait()
            pltpu.async_copy(g.at[row_v.at[nxt]], bufs[k], gsems[k])
        return 0

    lax.fori_loop(0, NCHUNK // _DEPTH - 1, body, 0)
    for k in range(_DEPTH):
        ch = NCHUNK - _DEPTH + k
        pltpu.make_async_copy(g.at[row_v.at[ch]], bufs[k], gsems[k]).wait()
        pltpu.async_copy(bufs[k], acc.at[col_v.at[ch]], ssems[k], add=True)
    for k in range(_DEPTH):
        pltpu.make_async_copy(bufs[k], acc.at[col_v.at[NCHUNK - _DEPTH + k]],
                              ssems[k]).wait()
    plsc.subcore_barrier()
    pltpu.sync_copy(acc.at[pl.ds(s * nz, nz)], out.at[c].at[pl.ds(s * nz, nz)])


@functools.partial(
    pl.kernel,
    mesh=_mesh,
    compiler_params=_sc_params,
    out_type=[
        jax.ShapeDtypeStruct((NC, N_PAD), jnp.float32),
        jax.ShapeDtypeStruct((N_PAD,), jnp.float32),
    ],
    scratch_types=[
        pltpu.VMEM((E_TILE,), jnp.int32),
        pltpu.VMEM((E_TILE,), jnp.int32),
        pltpu.VMEM((N_PAD,), jnp.float32),
        pltpu.VMEM((N_PAD,), jnp.float32),
        pltpu.VMEM((N_PAD,), jnp.float32),
        pltpu.VMEM((N_ROWS, 128), jnp.int32),
        pltpu.VMEM((N_ROWS, 128), jnp.int32),
        pltpu.VMEM_SHARED((N_PAD,), jnp.float32),
        pltpu.SemaphoreType.DMA,
        pltpu.SemaphoreType.DMA,
    ],
)
def _sc_agg1(rowf, colf, g7f, zeros1d, ident, ident64, out, g7d, row_v,
             col_v, g_local, loc_a, loc_b, idx_v, id64_v, acc, sem_r, sem_g):
    c = lax.axis_index("c")
    s = lax.axis_index("s")
    pltpu.sync_copy(rowf.at[c].at[s], row_v)
    pltpu.sync_copy(colf.at[c].at[s], col_v)
    pltpu.sync_copy(ident, idx_v)
    pltpu.sync_copy(ident64, id64_v)
    pltpu.sync_copy(zeros1d, loc_a)
    pltpu.sync_copy(zeros1d, loc_b)
    pltpu.sync_copy(zeros1d.at[pl.ds(s * NZ, NZ)], acc.at[pl.ds(s * NZ, NZ)])

    def stage(j, _):
        pltpu.async_copy(g7f.at[id64_v.at[j]],
                         g_local.at[pl.ds(j * 128, 128)], sem_g)
        return 0

    lax.fori_loop(0, N_ROWS, stage, 0)

    def stage_wait(j, _):
        pltpu.make_async_copy(g7f.at[id64_v.at[j]],
                              g_local.at[pl.ds(j * 128, 128)], sem_g).wait()
        return 0

    lax.fori_loop(0, N_ROWS, stage_wait, 0)

    @pl.when(c == 0)
    def _g7d():
        pltpu.sync_copy(g_local.at[pl.ds(s * NZ, NZ)],
                        g7d.at[pl.ds(s * NZ, NZ)])

    def body(i, _):
        ir = row_v[pl.ds(2 * i * 16, 16)]
        vals = plsc.load_gather(g_local, [ir])
        ic = col_v[pl.ds(2 * i * 16, 16)]
        plsc.addupdate_scatter(loc_a, [ic], vals)
        ir2 = row_v[pl.ds((2 * i + 1) * 16, 16)]
        vals2 = plsc.load_gather(g_local, [ir2])
        ic2 = col_v[pl.ds((2 * i + 1) * 16, 16)]
        plsc.addupdate_scatter(loc_b, [ic2], vals2)
        return 0

    n2 = E_TILE // 32
    lax.fori_loop(0, n2, body, 0)
    ir = row_v[pl.ds(2 * n2 * 16, 16)]
    vals = plsc.load_gather(g_local, [ir])
    ic = col_v[pl.ds(2 * n2 * 16, 16)]
    plsc.addupdate_scatter(loc_a, [ic], vals)
    plsc.subcore_barrier()

    def red(j, _):
        pltpu.async_copy(loc_a.at[pl.ds(j * 128, 128)],
                         acc.at[idx_v.at[j]], sem_r, add=True)
        pltpu.async_copy(loc_b.at[pl.ds(j * 128, 128)],
                         acc.at[idx_v.at[j]], sem_r, add=True)
        return 0

    lax.fori_loop(0, N_ROWS, red, 0)

    def red_wait(j, _):
        pltpu.make_async_copy(loc_a.at[pl.ds(j * 128, 128)],
                              acc.at[idx_v.at[j]], sem_r).wait()
        pltpu.make_async_copy(loc_b.at[pl.ds(j * 128, 128)],
                              acc.at[idx_v.at[j]], sem_r).wait()
        return 0

    lax.fori_loop(0, N_ROWS, red_wait, 0)
    plsc.subcore_barrier()
    pltpu.sync_copy(acc.at[pl.ds(s * NZ, NZ)], out.at[c].at[pl.ds(s * NZ, NZ)])


_R = 5120


_RP = _R // 2


def _tc_first(x_pk, dpk, W2_i):
    def body(dpk_ref, x_ref, w_ref, dpki_ref, g_ref):
        dinv = lax.rsqrt(1.0 + dpk_ref[0] + dpk_ref[1])
        dpki_ref[...] = dinv
        g_ref[...] = dinv * jnp.dot(x_ref[...], w_ref[...],
                                    preferred_element_type=jnp.float32)

    return pl.pallas_call(
        body,
        grid=(N_PAD // _R,),
        in_specs=[
            pl.BlockSpec((NC, _RP, 128), lambda i: (0, i, 0)),
            pl.BlockSpec((_RP, 256), lambda i: (i, 0)),
            pl.BlockSpec((256, 128), lambda i: (0, 0)),
        ],
        out_specs=[
            pl.BlockSpec((_RP, 128), lambda i: (i, 0)),
            pl.BlockSpec((_RP, 128), lambda i: (i, 0)),
        ],
        out_shape=[
            jax.ShapeDtypeStruct((N_PAD // 2, 128), jnp.float32),
            jax.ShapeDtypeStruct((N_PAD // 2, 128), jnp.float32),
        ],
    )(dpk, x_pk, W2_i)


def _tc_layer_pk(p_pk, dinv_pk, b2, W2):
    def body(p_ref, dinv_ref, b_ref, w_ref, o_ref):
        dinv = dinv_ref[...]
        h = jnp.maximum(dinv * (p_ref[0] + p_ref[1]) + b_ref[...], 0.0)
        o_ref[...] = dinv * jnp.dot(h, w_ref[...],
                                    preferred_element_type=jnp.float32)

    return pl.pallas_call(
        body,
        grid=(N_PAD // _R,),
        in_specs=[
            pl.BlockSpec((NC, _RP, 128), lambda i: (0, i, 0)),
            pl.BlockSpec((_RP, 128), lambda i: (i, 0)),
            pl.BlockSpec((1, 128), lambda i: (0, 0)),
            pl.BlockSpec((128, 128), lambda i: (0, 0)),
        ],
        out_specs=pl.BlockSpec((_RP, 128), lambda i: (i, 0)),
        out_shape=jax.ShapeDtypeStruct((N_PAD // 2, 128), jnp.float32),
    )(p_pk, dinv_pk, b2, W2)


def _tc_final(q, g7d, dn, b_o):
    def body(q_ref, g_ref, d_ref, b_ref, o_ref):
        dinv = lax.rsqrt(1.0 + d_ref[0] + d_ref[1])
        o_ref[...] = dinv * (q_ref[0] + q_ref[1] + g_ref[...]) + b_ref[...]

    return pl.pallas_call(
        body,
        out_shape=jax.ShapeDtypeStruct((N_ROWS, 128), jnp.float32),
    )(q, g7d, dn, b_o)


def kernel(x, edge_index, W_i, b_i, Wh, bh, W_o, b_o):
    row = edge_index[0].astype(jnp.int32)
    col = edge_index[1].astype(jnp.int32)
    row4 = row.reshape(NC, NS, NCHUNK, CHUNK)
    col4 = col.reshape(NC, NS, NCHUNK, CHUNK)
    rowf = row.reshape(NC, NS, E_TILE)
    colf = col.reshape(NC, NS, E_TILE)

    zeros1d = jnp.zeros((N_PAD,), jnp.float32)
    zeros64 = jnp.zeros((N_PAD, 64), jnp.float32)
    ident = jnp.arange(N_PAD, dtype=jnp.int32).reshape(N_ROWS, 128)
    x_pad = jnp.zeros((N_PAD, 128), jnp.float32).at[:N_NODES].set(x)

    def blockdiag2(W):
        din, dout = W.shape
        W2 = jnp.zeros((2 * din, 2 * dout), jnp.float32)
        return W2.at[:din, :dout].set(W).at[din:, dout:].set(W)

    dn_lin, dpk_flat = _sc_degree(colf, zeros1d, ident)
    dpk = dpk_flat.reshape(NC, N_PAD // 2, 128)
    dinv_pk, g = _tc_first(x_pad.reshape(N_PAD // 2, 256), dpk,
                           blockdiag2(W_i))

    for k in range(6):
        b = b_i if k == 0 else bh[k - 1]
        p = _sc_agg64(row4, col4, g.reshape(N_PAD, 64), zeros64)
        g = _tc_layer_pk(p.reshape(NC, N_PAD // 2, 128), dinv_pk,
                         jnp.concatenate([b, b]).reshape(1, 128),
                         blockdiag2(Wh[k]))

    W2_o = jnp.zeros((128, 128), jnp.float32)
    W2_o = W2_o.at[:64, 0].set(W_o[:, 0]).at[64:, 64].set(W_o[:, 0])
    b7 = bh[5]
    p = _sc_agg64(row4, col4, g.reshape(N_PAD, 64), zeros64)
    g7_pk = _tc_layer_pk(p.reshape(NC, N_PAD // 2, 128), dinv_pk,
                         jnp.concatenate([b7, b7]).reshape(1, 128), W2_o)

    ident64 = (jnp.arange(N_PAD, dtype=jnp.int32) * 64).reshape(N_ROWS, 128)
    q, g7d = _sc_agg1(rowf, colf, g7_pk.reshape(N_PAD * 64), zeros1d, ident,
                      ident64)
    out = _tc_final(q.reshape(NC, N_ROWS, 128), g7d.reshape(N_ROWS, 128),
                    dn_lin.reshape(NC, N_ROWS, 128),
                    jnp.broadcast_to(b_o.reshape(1, 1), (N_ROWS, 128)))
    return out.reshape(N_PAD, 1)[:N_NODES]


# speedup vs baseline: 42.8111x; 1.0006x over previous
import functools

import jax
import jax.numpy as jnp
from jax import lax
from jax.experimental import pallas as pl
from jax.experimental.pallas import tpu as pltpu
from jax.experimental.pallas import tpu_sc as plsc

N_NODES = 10000
N_PAD = 10240
N_ROWS = 80
E = 320000
NC = 2
NS = 16
E_SC = E // NC
E_TILE = E_SC // NS
CHUNK = 125
NCHUNK = E_TILE // CHUNK

_mesh = plsc.VectorSubcoreMesh(core_axis_name="c", subcore_axis_name="s")
_sc_params = pltpu.CompilerParams(needs_layout_passes=False,
                                  use_tc_tiling_on_sc=False)


NZ = N_PAD // NS


@functools.partial(
    pl.kernel,
    mesh=_mesh,
    compiler_params=_sc_params,
    out_type=[
        jax.ShapeDtypeStruct((NC, N_PAD), jnp.float32),
        jax.ShapeDtypeStruct((NC, N_PAD * 64), jnp.float32),
    ],
    scratch_types=[
        pltpu.VMEM((E_TILE,), jnp.int32),
        pltpu.VMEM((N_PAD,), jnp.float32),
        pltpu.VMEM((N_PAD,), jnp.float32),
        pltpu.VMEM((N_ROWS, 128), jnp.int32),
        pltpu.VMEM_SHARED((N_PAD,), jnp.float32),
        pltpu.VMEM((NZ,), jnp.float32),
        pltpu.VMEM((NZ * 64,), jnp.float32),
        pltpu.SemaphoreType.DMA,
    ],
)
def _sc_degree(colf, zeros1d, ident, out, out_pk, col_v, loc_a, loc_b, idx_v,
               acc, tmp, local_pk, sem_r):
    c = lax.axis_index("c")
    s = lax.axis_index("s")
    pltpu.sync_copy(colf.at[c].at[s], col_v)
    pltpu.sync_copy(zeros1d, loc_a)
    pltpu.sync_copy(zeros1d, loc_b)
    pltpu.sync_copy(ident, idx_v)
    pltpu.sync_copy(zeros1d.at[pl.ds(s * NZ, NZ)], acc.at[pl.ds(s * NZ, NZ)])
    ones = jnp.full((16,), 1.0, jnp.float32)

    def body(i, _):
        ic = col_v[pl.ds(2 * i * 16, 16)]
        plsc.addupdate_scatter(loc_a, [ic], ones)
        ic2 = col_v[pl.ds((2 * i + 1) * 16, 16)]
        plsc.addupdate_scatter(loc_b, [ic2], ones)
        return 0

    n2 = E_TILE // 32
    lax.fori_loop(0, n2, body, 0)
    ic = col_v[pl.ds(2 * n2 * 16, 16)]
    plsc.addupdate_scatter(loc_a, [ic], ones)
    plsc.subcore_barrier()

    def red(j, _):
        pltpu.async_copy(loc_a.at[pl.ds(j * 128, 128)],
                         acc.at[idx_v.at[j]], sem_r, add=True)
        pltpu.async_copy(loc_b.at[pl.ds(j * 128, 128)],
                         acc.at[idx_v.at[j]], sem_r, add=True)
        return 0

    lax.fori_loop(0, N_ROWS, red, 0)

    def red_wait(j, _):
        pltpu.make_async_copy(loc_a.at[pl.ds(j * 128, 128)],
                              acc.at[idx_v.at[j]], sem_r).wait()
        pltpu.make_async_copy(loc_b.at[pl.ds(j * 128, 128)],
                              acc.at[idx_v.at[j]], sem_r).wait()
        return 0

    lax.fori_loop(0, N_ROWS, red_wait, 0)
    plsc.subcore_barrier()
    pltpu.sync_copy(acc.at[pl.ds(s * NZ, NZ)], out.at[c].at[pl.ds(s * NZ, NZ)])
    pltpu.sync_copy(acc.at[pl.ds(s * NZ, NZ)], tmp)

    def rep(i, _):
        v16 = tmp[pl.ds(i * 16, 16)]
        for u in range(16):
            vec = jnp.full((16,), v16[u], jnp.float32)
            for k in range(4):
                local_pk[pl.ds((i * 16 + u) * 64 + k * 16, 16)] = vec
        return 0

    lax.fori_loop(0, NZ // 16, rep, 0)
    pltpu.sync_copy(local_pk, out_pk.at[c].at[pl.ds(s * NZ * 64, NZ * 64)])


_DEPTH = 8


@functools.partial(
    pl.kernel,
    mesh=_mesh,
    compiler_params=_sc_params,
    out_type=jax.ShapeDtypeStruct((NC, N_PAD, 64), jnp.float32),
    scratch_types=(
        [pltpu.VMEM((NCHUNK, CHUNK), jnp.int32)] * 2
        + [pltpu.VMEM((CHUNK, 64), jnp.float32)] * _DEPTH
        + [pltpu.VMEM_SHARED((N_PAD, 64), jnp.float32)]
        + [pltpu.SemaphoreType.DMA] * (2 * _DEPTH)
    ),
)
def _sc_agg64(row4, col4, g, zeros64, out, row_v, col_v, *rest):
    bufs = rest[:_DEPTH]
    acc = rest[_DEPTH]
    gsems = rest[_DEPTH + 1:2 * _DEPTH + 1]
    ssems = rest[2 * _DEPTH + 1:]
    c = lax.axis_index("c")
    s = lax.axis_index("s")
    pltpu.sync_copy(row4.at[c].at[s], row_v)
    pltpu.sync_copy(col4.at[c].at[s], col_v)
    nz = N_PAD // NS

    @pl.when(c == 0)
    def _init_g():
        pltpu.sync_copy(g.at[pl.ds(s * nz, nz)], acc.at[pl.ds(s * nz, nz)])

    @pl.when(c != 0)
    def _init_z():
        pltpu.sync_copy(zeros64.at[pl.ds(s * nz, nz)],
                        acc.at[pl.ds(s * nz, nz)])

    plsc.subcore_barrier()

    for k in range(_DEPTH):
        pltpu.async_copy(g.at[row_v.at[k]], bufs[k], gsems[k])

    def body(j, _):
        for k in range(_DEPTH):
            ch = _DEPTH * j + k
            pltpu.make_async_copy(g.at[row_v.at[ch]], bufs[k], gsems[k]).wait()
            pltpu.async_copy(bufs[k], acc.at[col_v.at[ch]], ssems[k],
                             add=True)
        for k in range(_DEPTH):
            nxt = _DEPTH * j + _DEPTH + k
            pltpu.make_async_copy(bufs[k], acc.at[col_v.at[_DEPTH * j + k]],
                                  ssems[k]).wait()
            pltpu.async_copy(g.at[row_v.at[nxt]], bufs[k], gsems[k])
        return 0

    lax.fori_loop(0, NCHUNK // _DEPTH - 1, body, 0)
    for k in range(_DEPTH):
        ch = NCHUNK - _DEPTH + k
        pltpu.make_async_copy(g.at[row_v.at[ch]], bufs[k], gsems[k]).wait()
        pltpu.async_copy(bufs[k], acc.at[col_v.at[ch]], ssems[k], add=True)
    for k in range(_DEPTH):
        pltpu.make_async_copy(bufs[k], acc.at[col_v.at[NCHUNK - _DEPTH + k]],
                              ssems[k]).wait()
    plsc.subcore_barrier()
    pltpu.sync_copy(acc.at[pl.ds(s * nz, nz)], out.at[c].at[pl.ds(s * nz, nz)])


@functools.partial(
    pl.kernel,
    mesh=_mesh,
    compiler_params=_sc_params,
    out_type=[
        jax.ShapeDtypeStruct((NC, N_PAD), jnp.float32),
        jax.ShapeDtypeStruct((N_PAD,), jnp.float32),
    ],
    scratch_types=[
        pltpu.VMEM((E_TILE,), jnp.int32),
        pltpu.VMEM((E_TILE,), jnp.int32),
        pltpu.VMEM((N_PAD,), jnp.float32),
        pltpu.VMEM((N_PAD,), jnp.float32),
        pltpu.VMEM((N_PAD,), jnp.float32),
        pltpu.VMEM((N_ROWS, 128), jnp.int32),
        pltpu.VMEM((N_ROWS, 128), jnp.int32),
        pltpu.VMEM_SHARED((N_PAD,), jnp.float32),
        pltpu.SemaphoreType.DMA,
        pltpu.SemaphoreType.DMA,
    ],
)
def _sc_agg1(rowf, colf, g7f, zeros1d, ident, ident64, out, g7d, row_v,
             col_v, g_local, loc_a, loc_b, idx_v, id64_v, acc, sem_r, sem_g):
    c = lax.axis_index("c")
    s = lax.axis_index("s")
    pltpu.sync_copy(rowf.at[c].at[s], row_v)
    pltpu.sync_copy(colf.at[c].at[s], col_v)
    pltpu.sync_copy(ident, idx_v)
    pltpu.sync_copy(ident64, id64_v)
    pltpu.sync_copy(zeros1d, loc_a)
    pltpu.sync_copy(zeros1d, loc_b)
    pltpu.sync_copy(zeros1d.at[pl.ds(s * NZ, NZ)], acc.at[pl.ds(s * NZ, NZ)])

    def stage(j, _):
        pltpu.async_copy(g7f.at[id64_v.at[j]],
                         g_local.at[pl.ds(j * 128, 128)], sem_g)
        return 0

    lax.fori_loop(0, N_ROWS, stage, 0)

    def stage_wait(j, _):
        pltpu.make_async_copy(g7f.at[id64_v.at[j]],
                              g_local.at[pl.ds(j * 128, 128)], sem_g).wait()
        return 0

    lax.fori_loop(0, N_ROWS, stage_wait, 0)

    @pl.when(c == 0)
    def _g7d():
        pltpu.sync_copy(g_local.at[pl.ds(s * NZ, NZ)],
                        g7d.at[pl.ds(s * NZ, NZ)])

    def body(i, _):
        ir = row_v[pl.ds(2 * i * 16, 16)]
        vals = plsc.load_gather(g_local, [ir])
        ic = col_v[pl.ds(2 * i * 16, 16)]
        plsc.addupdate_scatter(loc_a, [ic], vals)
        ir2 = row_v[pl.ds((2 * i + 1) * 16, 16)]
        vals2 = plsc.load_gather(g_local, [ir2])
        ic2 = col_v[pl.ds((2 * i + 1) * 16, 16)]
        plsc.addupdate_scatter(loc_b, [ic2], vals2)
        return 0

    n2 = E_TILE // 32
    lax.fori_loop(0, n2, body, 0)
    ir = row_v[pl.ds(2 * n2 * 16, 16)]
    vals = plsc.load_gather(g_local, [ir])
    ic = col_v[pl.ds(2 * n2 * 16, 16)]
    plsc.addupdate_scatter(loc_a, [ic], vals)
    plsc.subcore_barrier()

    def red(j, _):
        pltpu.async_copy(loc_a.at[pl.ds(j * 128, 128)],
                         acc.at[idx_v.at[j]], sem_r, add=True)
        pltpu.async_copy(loc_b.at[pl.ds(j * 128, 128)],
                         acc.at[idx_v.at[j]], sem_r, add=True)
        return 0

    lax.fori_loop(0, N_ROWS, red, 0)

    def red_wait(j, _):
        pltpu.make_async_copy(loc_a.at[pl.ds(j * 128, 128)],
                              acc.at[idx_v.at[j]], sem_r).wait()
        pltpu.make_async_copy(loc_b.at[pl.ds(j * 128, 128)],
                              acc.at[idx_v.at[j]], sem_r).wait()
        return 0

    lax.fori_loop(0, N_ROWS, red_wait, 0)
    plsc.subcore_barrier()
    pltpu.sync_copy(acc.at[pl.ds(s * NZ, NZ)], out.at[c].at[pl.ds(s * NZ, NZ)])


_R = 5120


_RP = _R // 2


def _tc_first(x_pk, dpk, W2_i):
    def body(dpk_ref, x_ref, w_ref, dpki_ref, g_ref):
        dinv = lax.rsqrt(1.0 + dpk_ref[0] + dpk_ref[1])
        dpki_ref[...] = dinv
        g_ref[...] = dinv * jnp.dot(x_ref[...], w_ref[...],
                                    preferred_element_type=jnp.float32)

    return pl.pallas_call(
        body,
        grid=(N_PAD // _R,),
        in_specs=[
            pl.BlockSpec((NC, _RP, 128), lambda i: (0, i, 0)),
            pl.BlockSpec((_RP, 256), lambda i: (i, 0)),
            pl.BlockSpec((256, 128), lambda i: (0, 0)),
        ],
        out_specs=[
            pl.BlockSpec((_RP, 128), lambda i: (i, 0)),
            pl.BlockSpec((_RP, 128), lambda i: (i, 0)),
        ],
        out_shape=[
            jax.ShapeDtypeStruct((N_PAD // 2, 128), jnp.float32),
            jax.ShapeDtypeStruct((N_PAD // 2, 128), jnp.float32),
        ],
    )(dpk, x_pk, W2_i)


def _tc_layer_pk(p_pk, dinv_pk, b2, W2):
    def body(p_ref, dinv_ref, b_ref, w_ref, o_ref):
        dinv = dinv_ref[...]
        h = jnp.maximum(dinv * (p_ref[0] + p_ref[1]) + b_ref[...], 0.0)
        o_ref[...] = dinv * jnp.dot(h, w_ref[...],
                                    preferred_element_type=jnp.float32)

    return pl.pallas_call(
        body,
        grid=(N_PAD // _R,),
        in_specs=[
            pl.BlockSpec((NC, _RP, 128), lambda i: (0, i, 0)),
            pl.BlockSpec((_RP, 128), lambda i: (i, 0)),
            pl.BlockSpec((1, 128), lambda i: (0, 0)),
            pl.BlockSpec((128, 128), lambda i: (0, 0)),
        ],
        out_specs=pl.BlockSpec((_RP, 128), lambda i: (i, 0)),
        out_shape=jax.ShapeDtypeStruct((N_PAD // 2, 128), jnp.float32),
    )(p_pk, dinv_pk, b2, W2)


def _tc_final(q, g7d, dn, b_o):
    def body(q_ref, g_ref, d_ref, b_ref, o_ref):
        dinv = lax.rsqrt(1.0 + d_ref[0] + d_ref[1])
        o_ref[...] = dinv * (q_ref[0] + q_ref[1] + g_ref[...]) + b_ref[...]

    return pl.pallas_call(
        body,
        out_shape=jax.ShapeDtypeStruct((N_ROWS, 128), jnp.float32),
    )(q, g7d, dn, b_o)


def kernel(x, edge_index, W_i, b_i, Wh, bh, W_o, b_o):
    row = edge_index[0].astype(jnp.int32)
    col = edge_index[1].astype(jnp.int32)
    row4 = row.reshape(NC, NS, NCHUNK, CHUNK)
    col4 = col.reshape(NC, NS, NCHUNK, CHUNK)
    rowf = row.reshape(NC, NS, E_TILE)
    colf = col.reshape(NC, NS, E_TILE)

    zeros1d = jnp.zeros((N_PAD,), jnp.float32)
    zeros64 = jnp.zeros((N_PAD, 64), jnp.float32)
    ident = jnp.arange(N_PAD, dtype=jnp.int32).reshape(N_ROWS, 128)
    x_pad = jnp.zeros((N_PAD, 128), jnp.float32).at[:N_NODES].set(x)

    def blockdiag2(W):
        din, dout = W.shape
        W2 = jnp.zeros((2 * din, 2 * dout), jnp.float32)
        return W2.at[:din, :dout].set(W).at[din:, dout:].set(W)

    dn_lin, dpk_flat = _sc_degree(colf, zeros1d, ident)
    dpk = dpk_flat.reshape(NC, N_PAD // 2, 128)
    dinv_pk, g = _tc_first(x_pad.reshape(N_PAD // 2, 256), dpk,
                           blockdiag2(W_i))

    for k in range(6):
        b = b_i if k == 0 else bh[k - 1]
        p = _sc_agg64(row4, col4, g.reshape(N_PAD, 64), zeros64)
        g = _tc_layer_pk(p.reshape(NC, N_PAD // 2, 128), dinv_pk,
                         jnp.concatenate([b, b]).reshape(1, 128),
                         blockdiag2(Wh[k]))

    W2_o = jnp.zeros((128, 128), jnp.float32)
    W2_o = W2_o.at[:64, 0].set(W_o[:, 0]).at[64:, 64].set(W_o[:, 0])
    b7 = bh[5]
    p = _sc_agg64(row4, col4, g.reshape(N_PAD, 64), zeros64)
    g7_pk = _tc_layer_pk(p.reshape(NC, N_PAD // 2, 128), dinv_pk,
                         jnp.concatenate([b7, b7]).reshape(1, 128), W2_o)

    ident64 = (jnp.arange(N_PAD, dtype=jnp.int32) * 64).reshape(N_ROWS, 128)
    q, g7d = _sc_agg1(rowf, colf, g7_pk.reshape(N_PAD * 64), zeros1d, ident,
                      ident64)
    out = _tc_final(q.reshape(NC, N_ROWS, 128), g7d.reshape(N_ROWS, 128),
                    dn_lin.reshape(NC, N_ROWS, 128),
                    jnp.broadcast_to(b_o.reshape(1, 1), (N_ROWS, 128)))
    return out.reshape(N_PAD, 1)[:N_NODES]
